# Initial kernel scaffold; baseline (speedup 1.0000x reference)
import jax
import jax.numpy as jnp
from jax.experimental import pallas as pl


def kernel(edge_index, edge_weight, text_ids, text_lens, label_ids, label_lens, graph_ids, embedding_Word, Emb_label, W_text_0, b_text_0, W_text_1, b_text_1, W_label_0, b_label_0, W_label_1, b_label_1, adap_W, fusion_W1, fusion_b1, fusion_Wo, fusion_bo):
    raise NotImplementedError("write your pallas kernel here")


# speedup vs baseline: 2.3060x; 2.3060x over previous
import functools

import jax
import jax.numpy as jnp
from jax import lax
from jax.experimental import pallas as pl
from jax.experimental.pallas import tpu as pltpu
from jax.experimental.pallas import tpu_sc as plsc

N = 10000
E = 320000
B = 100
D = 128
H = 128
OUT = 256
LT = 50
LL = 10

NC = 2
NS = 16
NW = NC * NS

NODE_BLK = 40
NUM_BLKS = N // NODE_BLK
EPT = E // NS
EK = 40
NCHUNK = EPT // EK
ROWS_PT = 624
INIT_CH = 208


def _full16(v):
    return jnp.full((16,), v, jnp.int32)


def _emb_body(ids_hbm, w_hbm, table_hbm, out_hbm, ids_v, w_v, rows_v, obuf, sem,
              nwords):
    c = lax.axis_index("c")
    s = lax.axis_index("s")
    w = s * NC + c

    nblk = 7 + jnp.where(w < NUM_BLKS - 7 * NW, 1, 0)

    def block_body(i, _):
        blk = w + i * NW
        nb0 = blk * NODE_BLK
        pltpu.sync_copy(ids_hbm.at[pl.ds(nb0, NODE_BLK)], ids_v)
        pltpu.sync_copy(w_hbm.at[pl.ds(nb0 * nwords, NODE_BLK * nwords)], w_v)

        def node_body(ln, _):
            pltpu.async_copy(table_hbm.at[ids_v.at[ln]], rows_v, sem).wait()
            wbase = ln * nwords
            wj = plsc.load_gather(w_v, [_full16(wbase)])
            acc = [rows_v[0, pl.ds(l * 16, 16)] * wj for l in range(8)]
            for j in range(1, nwords):
                wj = plsc.load_gather(w_v, [_full16(wbase + j)])
                for l in range(8):
                    acc[l] = acc[l] + rows_v[j, pl.ds(l * 16, 16)] * wj
            for l in range(8):
                obuf[ln, pl.ds(l * 16, 16)] = acc[l]
            return _

        lax.fori_loop(0, NODE_BLK, node_body, 0)
        pltpu.sync_copy(obuf, out_hbm.at[pl.ds(nb0, NODE_BLK)])
        return _

    lax.fori_loop(0, nblk, block_body, 0)


def _emb_kernel_fn(text_ids, wt, label_ids, wl, word_tab, label_tab,
                   out_t, out_l,
                   ids_tv, wt_v, rows_tv, ids_lv, wl_v, rows_lv, obuf, sem):
    _emb_body(text_ids, wt, word_tab, out_t, ids_tv, wt_v, rows_tv, obuf, sem,
              LT)
    _emb_body(label_ids, wl, label_tab, out_l, ids_lv, wl_v, rows_lv, obuf,
              sem, LL)


@functools.lru_cache(maxsize=None)
def _emb_call():
    return functools.partial(
    pl.kernel,
    out_type=[jax.ShapeDtypeStruct((N, D), jnp.float32),
              jax.ShapeDtypeStruct((N, D), jnp.float32)],
    mesh=plsc.VectorSubcoreMesh(core_axis_name="c", subcore_axis_name="s"),
    compiler_params=pltpu.CompilerParams(needs_layout_passes=False),
    scratch_types=[
        pltpu.VMEM((NODE_BLK, LT), jnp.int32),
        pltpu.VMEM((NODE_BLK * LT,), jnp.float32),
        pltpu.VMEM((LT, D), jnp.float32),
        pltpu.VMEM((NODE_BLK, LL), jnp.int32),
        pltpu.VMEM((NODE_BLK * LL,), jnp.float32),
        pltpu.VMEM((LL, D), jnp.float32),
        pltpu.VMEM((NODE_BLK, D), jnp.float32),
        pltpu.SemaphoreType.DMA,
    ],
    )(_emb_kernel_fn)


def _msg_branch(src_hbm, dst_hbm, ew_hbm, h_hbm, out_hbm, acc,
                srcb, dstb, ewb, rows_g, rows_s, ibuf, gsem):
    s = lax.axis_index("s")

    base = s * ROWS_PT
    for ci in range(ROWS_PT // INIT_CH):
        r0 = base + ci * INIT_CH
        pltpu.sync_copy(h_hbm.at[pl.ds(r0, INIT_CH)], ibuf)
        pltpu.sync_copy(ibuf, acc.at[pl.ds(r0, INIT_CH)])

    @pl.when(s == NS - 1)
    def _():
        r0 = NS * ROWS_PT
        pltpu.sync_copy(h_hbm.at[pl.ds(r0, N - NS * ROWS_PT)],
                        ibuf.at[pl.ds(0, N - NS * ROWS_PT)])
        pltpu.sync_copy(ibuf.at[pl.ds(0, N - NS * ROWS_PT)],
                        acc.at[pl.ds(r0, N - NS * ROWS_PT)])

    plsc.subcore_barrier()

    eb = s * EPT

    def load_idx(m, par):
        e0 = eb + m * EK
        pltpu.sync_copy(src_hbm.at[pl.ds(e0, EK)], srcb[par])
        pltpu.sync_copy(dst_hbm.at[pl.ds(e0, EK)], dstb[par])
        pltpu.sync_copy(ew_hbm.at[pl.ds(e0, EK)], ewb[par])

    load_idx(0, 0)
    h0 = pltpu.async_copy(h_hbm.at[srcb[0]], rows_g[0], gsem[0])

    def chunk_body(m, par):
        pltpu.make_async_copy(h_hbm.at[srcb[par]], rows_g[par],
                              gsem[par]).wait()

        @pl.when(m + 1 < NCHUNK)
        def _():
            load_idx(m + 1, 1 - par)
            pltpu.async_copy(h_hbm.at[srcb[1 - par]], rows_g[1 - par],
                             gsem[1 - par])

        for e in range(EK):
            wv = plsc.load_gather(ewb[par], [_full16(e)])
            for l in range(8):
                rows_s[par][e, pl.ds(l * 16, 16)] = (
                    rows_g[par][e, pl.ds(l * 16, 16)] * wv)

        pltpu.sync_copy(rows_s[par], acc.at[dstb[par]], add=True)

    def iter_body(it, _):
        chunk_body(it * 2, 0)
        chunk_body(it * 2 + 1, 1)
        return _

    lax.fori_loop(0, NCHUNK // 2, iter_body, 0)
    plsc.subcore_barrier()

    for ci in range(ROWS_PT // INIT_CH):
        r0 = base + ci * INIT_CH
        pltpu.sync_copy(acc.at[pl.ds(r0, INIT_CH)], ibuf)
        pltpu.sync_copy(ibuf, out_hbm.at[pl.ds(r0, INIT_CH)])

    @pl.when(s == NS - 1)
    def _():
        r0 = NS * ROWS_PT
        pltpu.sync_copy(acc.at[pl.ds(r0, N - NS * ROWS_PT)],
                        ibuf.at[pl.ds(0, N - NS * ROWS_PT)])
        pltpu.sync_copy(ibuf.at[pl.ds(0, N - NS * ROWS_PT)],
                        out_hbm.at[pl.ds(r0, N - NS * ROWS_PT)])


def _msg_kernel_fn(src, dst, ew, ht, hl, out_t, out_l,
                   acc, src0, src1, dst0, dst1, ew0, ew1,
                   rg0, rg1, rs0, rs1, ibuf, gsem0, gsem1):
    c = lax.axis_index("c")
    srcb = (src0, src1)
    dstb = (dst0, dst1)
    ewb = (ew0, ew1)
    rows_g = (rg0, rg1)
    rows_s = (rs0, rs1)
    gsem = (gsem0, gsem1)

    @pl.when(c == 0)
    def _():
        _msg_branch(src, dst, ew, ht, out_t, acc,
                    srcb, dstb, ewb, rows_g, rows_s, ibuf, gsem)

    @pl.when(c == 1)
    def _():
        _msg_branch(src, dst, ew, hl, out_l, acc,
                    srcb, dstb, ewb, rows_g, rows_s, ibuf, gsem)


@functools.lru_cache(maxsize=None)
def _msg_call():
    return functools.partial(
    pl.kernel,
    out_type=[jax.ShapeDtypeStruct((N, D), jnp.float32),
              jax.ShapeDtypeStruct((N, D), jnp.float32)],
    mesh=plsc.VectorSubcoreMesh(core_axis_name="c", subcore_axis_name="s"),
    compiler_params=pltpu.CompilerParams(needs_layout_passes=False),
    scratch_types=[
        pltpu.VMEM_SHARED((N, D), jnp.float32),
        pltpu.VMEM((EK,), jnp.int32),
        pltpu.VMEM((EK,), jnp.int32),
        pltpu.VMEM((EK,), jnp.int32),
        pltpu.VMEM((EK,), jnp.int32),
        pltpu.VMEM((EK,), jnp.float32),
        pltpu.VMEM((EK,), jnp.float32),
        pltpu.VMEM((EK, D), jnp.float32),
        pltpu.VMEM((EK, D), jnp.float32),
        pltpu.VMEM((EK, D), jnp.float32),
        pltpu.VMEM((EK, D), jnp.float32),
        pltpu.VMEM((INIT_CH, D), jnp.float32),
        pltpu.SemaphoreType.DMA,
        pltpu.SemaphoreType.DMA,
    ],
    )(_msg_kernel_fn)


C_BLK = 1000


def _combine_fn(at_ref, wt_ref, bt_ref, al_ref, wl_ref, bl_ref,
                ot_ref, ol_ref):
    xt = jnp.dot(at_ref[...], wt_ref[...],
                 preferred_element_type=jnp.float32) + bt_ref[...]
    ot_ref[...] = jnp.maximum(xt, 0.0)
    xl = jnp.dot(al_ref[...], wl_ref[...],
                 preferred_element_type=jnp.float32) + bl_ref[...]
    ol_ref[...] = jnp.maximum(xl, 0.0)


def _combine(acc_t, wtT, bt, acc_l, wlT, bl):
    blk = lambda i: (i, 0)
    fix = lambda i: (0, 0)
    return pl.pallas_call(
        _combine_fn,
        grid=(N // C_BLK,),
        in_specs=[
            pl.BlockSpec((C_BLK, D), blk),
            pl.BlockSpec((D, H), fix),
            pl.BlockSpec((1, H), fix),
            pl.BlockSpec((C_BLK, D), blk),
            pl.BlockSpec((D, H), fix),
            pl.BlockSpec((1, H), fix),
        ],
        out_specs=[
            pl.BlockSpec((C_BLK, H), blk),
            pl.BlockSpec((C_BLK, H), blk),
        ],
        out_shape=[jax.ShapeDtypeStruct((N, H), jnp.float32),
                   jax.ShapeDtypeStruct((N, H), jnp.float32)],
    )(acc_t, wtT, bt.reshape(1, H), acc_l, wlT, bl.reshape(1, H))


R_BLK = 1000
NR = N // R_BLK


def _readout_fn(ht_ref, hl_ref, g_ref, adap_ref, w1_ref, b1_ref,
                wo_ref, bo_ref, out_ref, acc_t, acc_l, cnt):
    i = pl.program_id(0)

    @pl.when(i == 0)
    def _():
        acc_t[...] = jnp.zeros_like(acc_t)
        acc_l[...] = jnp.zeros_like(acc_l)
        cnt[...] = jnp.zeros_like(cnt)

    g = g_ref[...]
    cols = lax.broadcasted_iota(jnp.int32, (R_BLK, 128), 1)
    onehot = (g == cols).astype(jnp.float32)

    contract = (((0,), (0,)), ((), ()))
    acc_t[...] += lax.dot_general(onehot, ht_ref[...], contract,
                                  preferred_element_type=jnp.float32)
    acc_l[...] += lax.dot_general(onehot, hl_ref[...], contract,
                                  preferred_element_type=jnp.float32)
    ones = jnp.ones((R_BLK, 1), jnp.float32)
    cnt[...] += lax.dot_general(onehot, ones, contract,
                                preferred_element_type=jnp.float32)

    @pl.when(i == NR - 1)
    def _():
        c = jnp.maximum(cnt[...], 1.0)
        r_t = acc_t[...] / c
        r_l = acc_l[...] / c
        a0 = adap_ref[0, 0]
        a1 = adap_ref[0, 1]
        adap_out = a0 * r_t + a1 * r_l
        fused = jnp.maximum(
            jnp.dot(adap_out, w1_ref[...],
                    preferred_element_type=jnp.float32) + b1_ref[...], 0.0)
        res = jnp.dot(fused, wo_ref[...],
                      preferred_element_type=jnp.float32) + bo_ref[...]
        out_ref[...] = res[:B, :]


def _readout(ht, hl, gids2, adap_W, w1T, b1, woT, bo):
    blk = lambda i: (i, 0)
    fix = lambda i: (0, 0)
    return pl.pallas_call(
        _readout_fn,
        grid=(NR,),
        in_specs=[
            pl.BlockSpec((R_BLK, H), blk),
            pl.BlockSpec((R_BLK, H), blk),
            pl.BlockSpec((R_BLK, 1), blk),
            pl.BlockSpec(memory_space=pltpu.SMEM),
            pl.BlockSpec((H, H), fix),
            pl.BlockSpec((1, H), fix),
            pl.BlockSpec((H, OUT), fix),
            pl.BlockSpec((1, OUT), fix),
        ],
        out_specs=pl.BlockSpec((B, OUT), fix),
        out_shape=jax.ShapeDtypeStruct((B, OUT), jnp.float32),
        scratch_shapes=[
            pltpu.VMEM((128, H), jnp.float32),
            pltpu.VMEM((128, H), jnp.float32),
            pltpu.VMEM((128, 1), jnp.float32),
        ],
    )(ht, hl, gids2, adap_W, w1T, b1.reshape(1, H), woT, bo.reshape(1, OUT))


def kernel(edge_index, edge_weight, text_ids, text_lens, label_ids,
           label_lens, graph_ids, embedding_Word, Emb_label,
           W_text_0, b_text_0, W_text_1, b_text_1,
           W_label_0, b_label_0, W_label_1, b_label_1,
           adap_W, fusion_W1, fusion_b1, fusion_Wo, fusion_bo):
    src = edge_index[0].astype(jnp.int32)
    dst = edge_index[1].astype(jnp.int32)
    ew = edge_weight.astype(jnp.float32)

    tl = text_lens.astype(jnp.float32)
    wt = jnp.where(jnp.arange(LT)[None, :] < text_lens[:, None],
                   1.0 / tl[:, None], 0.0).astype(jnp.float32)
    wl = jnp.where(jnp.arange(LL)[None, :] < label_lens[:, None],
                   1.0, 0.0).astype(jnp.float32)

    text_feat, label_feat = _emb_call()(
        text_ids.astype(jnp.int32), wt.reshape(-1),
        label_ids.astype(jnp.int32), wl.reshape(-1),
        embedding_Word, Emb_label)

    acc_t, acc_l = _msg_call()(src, dst, ew, text_feat, label_feat)
    h1t, h1l = _combine(acc_t, W_text_0.T, b_text_0,
                        acc_l, W_label_0.T, b_label_0)
    acc_t2, acc_l2 = _msg_call()(src, dst, ew, h1t, h1l)
    h2t, h2l = _combine(acc_t2, W_text_1.T, b_text_1,
                        acc_l2, W_label_1.T, b_label_1)

    res = _readout(h2t, h2l, graph_ids.astype(jnp.int32).reshape(N, 1),
                   adap_W, fusion_W1.T, fusion_b1, fusion_Wo.T, fusion_bo)
    return res


# speedup vs baseline: 4.6355x; 2.0102x over previous
import functools

import jax
import jax.numpy as jnp
from jax import lax
from jax.experimental import pallas as pl
from jax.experimental.pallas import tpu as pltpu
from jax.experimental.pallas import tpu_sc as plsc

N = 10000
E = 320000
B = 100
D = 128
H = 128
OUT = 256
LT = 50
LL = 10

NC = 2
NS = 16
NW = NC * NS

NODE_BLK = 40
NUM_BLKS = N // NODE_BLK
EPT = E // NS
EK = 40
NCHUNK = EPT // EK
ROWS_PT = 624
INIT_CH = 208


def _full16(v):
    return jnp.full((16,), v, jnp.int32)


def _emb_body(ids_hbm, w_hbm, table_hbm, out_hbm, ids_v, w_v, rows, obuf,
              sems, nwords):
    c = lax.axis_index("c")
    s = lax.axis_index("s")
    w = s * NC + c

    nblk = 7 + jnp.where(w < NUM_BLKS - 7 * NW, 1, 0)

    def block_body(i, _):
        blk = w + i * NW
        nb0 = blk * NODE_BLK
        pltpu.sync_copy(ids_hbm.at[pl.ds(nb0, NODE_BLK)], ids_v)
        pltpu.sync_copy(w_hbm.at[pl.ds(nb0 * nwords, NODE_BLK * nwords)], w_v)

        pltpu.async_copy(table_hbm.at[ids_v.at[0]], rows[0], sems[0])

        def node_body(ln, par):
            pltpu.make_async_copy(table_hbm.at[ids_v.at[ln]], rows[par],
                                  sems[par]).wait()

            @pl.when(ln + 1 < NODE_BLK)
            def _():
                pltpu.async_copy(table_hbm.at[ids_v.at[ln + 1]],
                                 rows[1 - par], sems[1 - par])

            wbase = ln * nwords
            wj = plsc.load_gather(w_v, [_full16(wbase)])
            acc = [rows[par][0, pl.ds(l * 16, 16)] * wj for l in range(8)]
            for j in range(1, nwords):
                wj = plsc.load_gather(w_v, [_full16(wbase + j)])
                for l in range(8):
                    acc[l] = acc[l] + rows[par][j, pl.ds(l * 16, 16)] * wj
            for l in range(8):
                obuf[ln, pl.ds(l * 16, 16)] = acc[l]

        def pair_body(p, _):
            node_body(p * 2, 0)
            node_body(p * 2 + 1, 1)
            return _

        lax.fori_loop(0, NODE_BLK // 2, pair_body, 0)
        pltpu.sync_copy(obuf, out_hbm.at[pl.ds(nb0, NODE_BLK)])
        return _

    lax.fori_loop(0, nblk, block_body, 0)


def _emb_kernel_fn(text_ids, wt, label_ids, wl, word_tab, label_tab,
                   out_t, out_l,
                   ids_tv, wt_v, rt0, rt1, ids_lv, wl_v, rl0, rl1, obuf,
                   sem0, sem1):
    _emb_body(text_ids, wt, word_tab, out_t, ids_tv, wt_v, (rt0, rt1), obuf,
              (sem0, sem1), LT)
    _emb_body(label_ids, wl, label_tab, out_l, ids_lv, wl_v, (rl0, rl1), obuf,
              (sem0, sem1), LL)


@functools.lru_cache(maxsize=None)
def _emb_call():
    return functools.partial(
    pl.kernel,
    out_type=[jax.ShapeDtypeStruct((N, D), jnp.float32),
              jax.ShapeDtypeStruct((N, D), jnp.float32)],
    mesh=plsc.VectorSubcoreMesh(core_axis_name="c", subcore_axis_name="s"),
    compiler_params=pltpu.CompilerParams(needs_layout_passes=False),
    scratch_types=[
        pltpu.VMEM((NODE_BLK, LT), jnp.int32),
        pltpu.VMEM((NODE_BLK * LT,), jnp.float32),
        pltpu.VMEM((LT, D), jnp.float32),
        pltpu.VMEM((LT, D), jnp.float32),
        pltpu.VMEM((NODE_BLK, LL), jnp.int32),
        pltpu.VMEM((NODE_BLK * LL,), jnp.float32),
        pltpu.VMEM((LL, D), jnp.float32),
        pltpu.VMEM((LL, D), jnp.float32),
        pltpu.VMEM((NODE_BLK, D), jnp.float32),
        pltpu.SemaphoreType.DMA,
        pltpu.SemaphoreType.DMA,
    ],
    )(_emb_kernel_fn)


NPH = 5
EPP = EPT // NPH
CPP = NCHUNK // NPH


def _msg_branch(src_hbm, dst4_hbm, ew_hbm, h_hbm, out_hbm, acc,
                src_flat, dst_loc, ew_flat, rows, gsem, ssem):
    s = lax.axis_index("s")

    base = s * ROWS_PT
    for ci in range(15):
        r0 = base + ci * 40
        pltpu.sync_copy(h_hbm.at[pl.ds(r0, 40)], rows[0])
        pltpu.sync_copy(rows[0], acc.at[pl.ds(r0, 40)])
    r0 = base + 600
    pltpu.sync_copy(h_hbm.at[pl.ds(r0, 24)], rows[0].at[pl.ds(0, 24)])
    pltpu.sync_copy(rows[0].at[pl.ds(0, 24)], acc.at[pl.ds(r0, 24)])

    @pl.when(s == NS - 1)
    def _():
        r0 = NS * ROWS_PT
        pltpu.sync_copy(h_hbm.at[pl.ds(r0, N - NS * ROWS_PT)],
                        rows[1].at[pl.ds(0, N - NS * ROWS_PT)])
        pltpu.sync_copy(rows[1].at[pl.ds(0, N - NS * ROWS_PT)],
                        acc.at[pl.ds(r0, N - NS * ROWS_PT)])

    plsc.subcore_barrier()

    dst_s = dst4_hbm.at[s]

    for ph in range(NPH):
        e0 = s * EPT + ph * EPP
        pltpu.sync_copy(src_hbm.at[pl.ds(e0, EPP)], src_flat)
        pltpu.sync_copy(ew_hbm.at[pl.ds(e0, EPP)], ew_flat)
        pltpu.sync_copy(dst_s.at[ph], dst_loc)

        def issue_gather(m, par):
            pltpu.async_copy(h_hbm.at[src_flat.at[pl.ds(m * EK, EK)]],
                             rows[par], gsem[par])

        issue_gather(0, 0)

        def chunk_body(m, par):
            @pl.when(m >= 1)
            def _():
                pltpu.make_async_copy(rows[1 - par],
                                      acc.at[dst_loc.at[m - 1]],
                                      ssem[1 - par]).wait()

            @pl.when(m + 1 < CPP)
            def _():
                issue_gather(m + 1, 1 - par)

            pltpu.make_async_copy(h_hbm.at[src_flat.at[pl.ds(m * EK, EK)]],
                                  rows[par], gsem[par]).wait()
            wb = m * EK

            def scale4(e4, _):
                e = e4 * 4
                for u in range(4):
                    wv = plsc.load_gather(
                        ew_flat, [jnp.full((16,), wb + e + u, jnp.int32)])
                    for l in range(8):
                        rows[par][e + u, pl.ds(l * 16, 16)] = (
                            rows[par][e + u, pl.ds(l * 16, 16)] * wv)
                return _

            lax.fori_loop(0, EK // 4, scale4, 0)

            pltpu.async_copy(rows[par], acc.at[dst_loc.at[m]], ssem[par],
                             add=True)

        def iter_body(it, _):
            chunk_body(it * 2, 0)
            chunk_body(it * 2 + 1, 1)
            return _

        lax.fori_loop(0, CPP // 2, iter_body, 0)
        pltpu.make_async_copy(rows[1], acc.at[dst_loc.at[CPP - 1]],
                              ssem[1]).wait()

    plsc.subcore_barrier()

    for ci in range(15):
        r0 = base + ci * 40
        pltpu.sync_copy(acc.at[pl.ds(r0, 40)], rows[0])
        pltpu.sync_copy(rows[0], out_hbm.at[pl.ds(r0, 40)])
    r0 = base + 600
    pltpu.sync_copy(acc.at[pl.ds(r0, 24)], rows[0].at[pl.ds(0, 24)])
    pltpu.sync_copy(rows[0].at[pl.ds(0, 24)], out_hbm.at[pl.ds(r0, 24)])

    @pl.when(s == NS - 1)
    def _():
        r0 = NS * ROWS_PT
        pltpu.sync_copy(acc.at[pl.ds(r0, N - NS * ROWS_PT)],
                        rows[1].at[pl.ds(0, N - NS * ROWS_PT)])
        pltpu.sync_copy(rows[1].at[pl.ds(0, N - NS * ROWS_PT)],
                        out_hbm.at[pl.ds(r0, N - NS * ROWS_PT)])


def _msg_kernel_fn(src, dst4, ew, ht, hl, out_t, out_l,
                   acc, src_flat, dst_loc, ew_flat, rg0, rg1,
                   gsem0, gsem1, ssem0, ssem1):
    c = lax.axis_index("c")
    rows = (rg0, rg1)
    gsem = (gsem0, gsem1)
    ssem = (ssem0, ssem1)

    @pl.when(c == 0)
    def _():
        _msg_branch(src, dst4, ew, ht, out_t, acc,
                    src_flat, dst_loc, ew_flat, rows, gsem, ssem)

    @pl.when(c == 1)
    def _():
        _msg_branch(src, dst4, ew, hl, out_l, acc,
                    src_flat, dst_loc, ew_flat, rows, gsem, ssem)


@functools.lru_cache(maxsize=None)
def _msg_call():
    return functools.partial(
    pl.kernel,
    out_type=[jax.ShapeDtypeStruct((N, D), jnp.float32),
              jax.ShapeDtypeStruct((N, D), jnp.float32)],
    mesh=plsc.VectorSubcoreMesh(core_axis_name="c", subcore_axis_name="s"),
    compiler_params=pltpu.CompilerParams(needs_layout_passes=False),
    scratch_types=[
        pltpu.VMEM_SHARED((N, D), jnp.float32),
        pltpu.VMEM((EPP,), jnp.int32),
        pltpu.VMEM((CPP, EK), jnp.int32),
        pltpu.VMEM((EPP,), jnp.float32),
        pltpu.VMEM((EK, D), jnp.float32),
        pltpu.VMEM((EK, D), jnp.float32),
        pltpu.SemaphoreType.DMA,
        pltpu.SemaphoreType.DMA,
        pltpu.SemaphoreType.DMA,
        pltpu.SemaphoreType.DMA,
    ],
    )(_msg_kernel_fn)


C_BLK = 1000


def _combine_fn(at_ref, wt_ref, bt_ref, al_ref, wl_ref, bl_ref,
                ot_ref, ol_ref):
    xt = jnp.dot(at_ref[...], wt_ref[...],
                 preferred_element_type=jnp.float32) + bt_ref[...]
    ot_ref[...] = jnp.maximum(xt, 0.0)
    xl = jnp.dot(al_ref[...], wl_ref[...],
                 preferred_element_type=jnp.float32) + bl_ref[...]
    ol_ref[...] = jnp.maximum(xl, 0.0)


def _combine(acc_t, wtT, bt, acc_l, wlT, bl):
    blk = lambda i: (i, 0)
    fix = lambda i: (0, 0)
    return pl.pallas_call(
        _combine_fn,
        grid=(N // C_BLK,),
        in_specs=[
            pl.BlockSpec((C_BLK, D), blk),
            pl.BlockSpec((D, H), fix),
            pl.BlockSpec((1, H), fix),
            pl.BlockSpec((C_BLK, D), blk),
            pl.BlockSpec((D, H), fix),
            pl.BlockSpec((1, H), fix),
        ],
        out_specs=[
            pl.BlockSpec((C_BLK, H), blk),
            pl.BlockSpec((C_BLK, H), blk),
        ],
        out_shape=[jax.ShapeDtypeStruct((N, H), jnp.float32),
                   jax.ShapeDtypeStruct((N, H), jnp.float32)],
    )(acc_t, wtT, bt.reshape(1, H), acc_l, wlT, bl.reshape(1, H))


R_BLK = 1000
NR = N // R_BLK


def _readout_fn(ht_ref, hl_ref, g_ref, adap_ref, w1_ref, b1_ref,
                wo_ref, bo_ref, out_ref, acc_t, acc_l, cnt):
    i = pl.program_id(0)

    @pl.when(i == 0)
    def _():
        acc_t[...] = jnp.zeros_like(acc_t)
        acc_l[...] = jnp.zeros_like(acc_l)
        cnt[...] = jnp.zeros_like(cnt)

    g = g_ref[...]
    cols = lax.broadcasted_iota(jnp.int32, (R_BLK, 128), 1)
    onehot = (g == cols).astype(jnp.float32)

    contract = (((0,), (0,)), ((), ()))
    acc_t[...] += lax.dot_general(onehot, ht_ref[...], contract,
                                  preferred_element_type=jnp.float32)
    acc_l[...] += lax.dot_general(onehot, hl_ref[...], contract,
                                  preferred_element_type=jnp.float32)
    ones = jnp.ones((R_BLK, 1), jnp.float32)
    cnt[...] += lax.dot_general(onehot, ones, contract,
                                preferred_element_type=jnp.float32)

    @pl.when(i == NR - 1)
    def _():
        c = jnp.maximum(cnt[...], 1.0)
        r_t = acc_t[...] / c
        r_l = acc_l[...] / c
        a0 = adap_ref[0, 0]
        a1 = adap_ref[0, 1]
        adap_out = a0 * r_t + a1 * r_l
        fused = jnp.maximum(
            jnp.dot(adap_out, w1_ref[...],
                    preferred_element_type=jnp.float32) + b1_ref[...], 0.0)
        res = jnp.dot(fused, wo_ref[...],
                      preferred_element_type=jnp.float32) + bo_ref[...]
        out_ref[...] = res[:B, :]


def _readout(ht, hl, gids2, adap_W, w1T, b1, woT, bo):
    blk = lambda i: (i, 0)
    fix = lambda i: (0, 0)
    return pl.pallas_call(
        _readout_fn,
        grid=(NR,),
        in_specs=[
            pl.BlockSpec((R_BLK, H), blk),
            pl.BlockSpec((R_BLK, H), blk),
            pl.BlockSpec((R_BLK, 1), blk),
            pl.BlockSpec(memory_space=pltpu.SMEM),
            pl.BlockSpec((H, H), fix),
            pl.BlockSpec((1, H), fix),
            pl.BlockSpec((H, OUT), fix),
            pl.BlockSpec((1, OUT), fix),
        ],
        out_specs=pl.BlockSpec((B, OUT), fix),
        out_shape=jax.ShapeDtypeStruct((B, OUT), jnp.float32),
        scratch_shapes=[
            pltpu.VMEM((128, H), jnp.float32),
            pltpu.VMEM((128, H), jnp.float32),
            pltpu.VMEM((128, 1), jnp.float32),
        ],
    )(ht, hl, gids2, adap_W, w1T, b1.reshape(1, H), woT, bo.reshape(1, OUT))


def kernel(edge_index, edge_weight, text_ids, text_lens, label_ids,
           label_lens, graph_ids, embedding_Word, Emb_label,
           W_text_0, b_text_0, W_text_1, b_text_1,
           W_label_0, b_label_0, W_label_1, b_label_1,
           adap_W, fusion_W1, fusion_b1, fusion_Wo, fusion_bo):
    src = edge_index[0].astype(jnp.int32)
    dst = edge_index[1].astype(jnp.int32)
    ew = edge_weight.astype(jnp.float32)

    tl = text_lens.astype(jnp.float32)
    wt = jnp.where(jnp.arange(LT)[None, :] < text_lens[:, None],
                   1.0 / tl[:, None], 0.0).astype(jnp.float32)
    wl = jnp.where(jnp.arange(LL)[None, :] < label_lens[:, None],
                   1.0, 0.0).astype(jnp.float32)

    text_feat, label_feat = _emb_call()(
        text_ids.astype(jnp.int32), wt.reshape(-1),
        label_ids.astype(jnp.int32), wl.reshape(-1),
        embedding_Word, Emb_label)

    dst4 = dst.reshape(NS, NPH, CPP, EK)
    acc_t, acc_l = _msg_call()(src, dst4, ew, text_feat, label_feat)
    h1t, h1l = _combine(acc_t, W_text_0.T, b_text_0,
                        acc_l, W_label_0.T, b_label_0)
    acc_t2, acc_l2 = _msg_call()(src, dst4, ew, h1t, h1l)
    h2t, h2l = _combine(acc_t2, W_text_1.T, b_text_1,
                        acc_l2, W_label_1.T, b_label_1)

    res = _readout(h2t, h2l, graph_ids.astype(jnp.int32).reshape(N, 1),
                   adap_W, fusion_W1.T, fusion_b1, fusion_Wo.T, fusion_bo)
    return res


# speedup vs baseline: 7.1728x; 1.5474x over previous
import functools

import jax
import jax.numpy as jnp
from jax import lax
from jax.experimental import pallas as pl
from jax.experimental.pallas import tpu as pltpu
from jax.experimental.pallas import tpu_sc as plsc

N = 10000
E = 320000
B = 100
D = 128
H = 128
OUT = 256
LT = 50
LL = 10

NC = 2
NS = 16
NW = NC * NS

NODE_BLK = 40
NUM_BLKS = N // NODE_BLK
EPT = E // NS
EK = 40
NCHUNK = EPT // EK
ROWS_PT = 624
INIT_CH = 208


def _full16(v):
    return jnp.full((16,), v, jnp.int32)


BN = 4
NBT = NODE_BLK // BN


def _emb_body(ids_hbm, w_hbm, table_hbm, out_hbm, ids_v, w_v, rows, obuf,
              sems, nwords):
    c = lax.axis_index("c")
    s = lax.axis_index("s")
    w = s * NC + c

    nblk = 7 + jnp.where(w < NUM_BLKS - 7 * NW, 1, 0)
    bw = BN * nwords

    def issue_gather(bt, par):
        pltpu.async_copy(table_hbm.at[ids_v.at[pl.ds(bt * bw, bw)]],
                         rows[par], sems[par])

    def block_body(i, _):
        blk = w + i * NW
        nb0 = blk * NODE_BLK
        pltpu.sync_copy(ids_hbm.at[pl.ds(nb0 * nwords, NODE_BLK * nwords)],
                        ids_v)
        pltpu.sync_copy(w_hbm.at[pl.ds(nb0 * nwords, NODE_BLK * nwords)], w_v)

        issue_gather(0, 0)

        def batch_body(bt, par):
            pltpu.make_async_copy(table_hbm.at[ids_v.at[pl.ds(bt * bw, bw)]],
                                  rows[par], sems[par]).wait()

            @pl.when(bt + 1 < NBT)
            def _():
                issue_gather(bt + 1, 1 - par)

            def node_body(u, _):
                wbase = (bt * BN + u) * nwords
                rbase = u * nwords
                wj = plsc.load_gather(w_v, [_full16(wbase)])
                acc = [rows[par][rbase, pl.ds(l * 16, 16)] * wj
                       for l in range(8)]
                for j in range(1, nwords):
                    wj = plsc.load_gather(w_v, [_full16(wbase + j)])
                    for l in range(8):
                        acc[l] = (acc[l]
                                  + rows[par][rbase + j, pl.ds(l * 16, 16)]
                                  * wj)
                ln = bt * BN + u
                for l in range(8):
                    obuf[ln, pl.ds(l * 16, 16)] = acc[l]
                return _

            lax.fori_loop(0, BN, node_body, 0)

        def pair_body(p, _):
            batch_body(p * 2, 0)
            batch_body(p * 2 + 1, 1)
            return _

        lax.fori_loop(0, NBT // 2, pair_body, 0)
        pltpu.sync_copy(obuf, out_hbm.at[pl.ds(nb0, NODE_BLK)])
        return _

    lax.fori_loop(0, nblk, block_body, 0)


def _emb_kernel_fn(text_ids, wt, label_ids, wl, word_tab, label_tab,
                   out_t, out_l,
                   ids_tv, wt_v, rt0, rt1, ids_lv, wl_v, rl0, rl1, obuf,
                   sem0, sem1):
    _emb_body(text_ids, wt, word_tab, out_t, ids_tv, wt_v, (rt0, rt1), obuf,
              (sem0, sem1), LT)
    _emb_body(label_ids, wl, label_tab, out_l, ids_lv, wl_v, (rl0, rl1), obuf,
              (sem0, sem1), LL)


@functools.lru_cache(maxsize=None)
def _emb_call():
    return functools.partial(
    pl.kernel,
    out_type=[jax.ShapeDtypeStruct((N, D), jnp.float32),
              jax.ShapeDtypeStruct((N, D), jnp.float32)],
    mesh=plsc.VectorSubcoreMesh(core_axis_name="c", subcore_axis_name="s"),
    compiler_params=pltpu.CompilerParams(needs_layout_passes=False),
    scratch_types=[
        pltpu.VMEM((NODE_BLK * LT,), jnp.int32),
        pltpu.VMEM((NODE_BLK * LT,), jnp.float32),
        pltpu.VMEM((BN * LT, D), jnp.float32),
        pltpu.VMEM((BN * LT, D), jnp.float32),
        pltpu.VMEM((NODE_BLK * LL,), jnp.int32),
        pltpu.VMEM((NODE_BLK * LL,), jnp.float32),
        pltpu.VMEM((BN * LL, D), jnp.float32),
        pltpu.VMEM((BN * LL, D), jnp.float32),
        pltpu.VMEM((NODE_BLK, D), jnp.float32),
        pltpu.SemaphoreType.DMA,
        pltpu.SemaphoreType.DMA,
    ],
    )(_emb_kernel_fn)


NPH = 5
EPP = EPT // NPH
CPP = NCHUNK // NPH
NBUF = 4


def _msg_branch(src_hbm, dst4_hbm, ew_hbm, h_hbm, out_hbm, acc,
                src_flat, dst_loc, ew_flat, rows, gsem, ssem):
    s = lax.axis_index("s")

    base = s * ROWS_PT
    for ci in range(15):
        r0 = base + ci * 40
        pltpu.sync_copy(h_hbm.at[pl.ds(r0, 40)], rows[0])
        pltpu.sync_copy(rows[0], acc.at[pl.ds(r0, 40)])
    r0 = base + 600
    pltpu.sync_copy(h_hbm.at[pl.ds(r0, 24)], rows[0].at[pl.ds(0, 24)])
    pltpu.sync_copy(rows[0].at[pl.ds(0, 24)], acc.at[pl.ds(r0, 24)])

    @pl.when(s == NS - 1)
    def _():
        r0 = NS * ROWS_PT
        pltpu.sync_copy(h_hbm.at[pl.ds(r0, N - NS * ROWS_PT)],
                        rows[1].at[pl.ds(0, N - NS * ROWS_PT)])
        pltpu.sync_copy(rows[1].at[pl.ds(0, N - NS * ROWS_PT)],
                        acc.at[pl.ds(r0, N - NS * ROWS_PT)])

    plsc.subcore_barrier()

    dst_s = dst4_hbm.at[s]

    def issue_gather(m, r):
        pltpu.async_copy(h_hbm.at[src_flat.at[pl.ds(m * EK, EK)]],
                         rows[r], gsem[r])

    def wait_gather(m, r):
        pltpu.make_async_copy(h_hbm.at[src_flat.at[pl.ds(m * EK, EK)]],
                              rows[r], gsem[r]).wait()

    def wait_scatter(m, r):
        pltpu.make_async_copy(rows[r], acc.at[dst_loc.at[m]], ssem[r]).wait()

    def chunk_body(m, r):
        @pl.when(m >= 2)
        def _():
            wait_scatter(m - 2, r ^ 2)

        @pl.when(m + 2 < CPP)
        def _():
            issue_gather(m + 2, r ^ 2)

        wait_gather(m, r)
        wb = m * EK

        def scale2(e2, _):
            e = e2 * 2
            for u in range(2):
                wv = plsc.load_gather(
                    ew_flat, [jnp.full((16,), wb + e + u, jnp.int32)])
                for l in range(8):
                    rows[r][e + u, pl.ds(l * 16, 16)] = (
                        rows[r][e + u, pl.ds(l * 16, 16)] * wv)
            return _

        lax.fori_loop(0, EK // 2, scale2, 0)

        pltpu.async_copy(rows[r], acc.at[dst_loc.at[m]], ssem[r], add=True)

    def phase_body(ph, _):
        e0 = s * EPT + ph * EPP
        pltpu.sync_copy(src_hbm.at[pl.ds(e0, EPP)], src_flat)
        pltpu.sync_copy(ew_hbm.at[pl.ds(e0, EPP)], ew_flat)
        pltpu.sync_copy(dst_s.at[ph], dst_loc)

        issue_gather(0, 0)
        issue_gather(1, 1)

        def iter_body(it, _):
            for u in range(NBUF):
                chunk_body(it * NBUF + u, u)
            return _

        lax.fori_loop(0, CPP // NBUF, iter_body, 0)
        wait_scatter(CPP - 2, (CPP - 2) % NBUF)
        wait_scatter(CPP - 1, (CPP - 1) % NBUF)
        return _

    lax.fori_loop(0, NPH, phase_body, 0)

    plsc.subcore_barrier()

    for ci in range(15):
        r0 = base + ci * 40
        pltpu.sync_copy(acc.at[pl.ds(r0, 40)], rows[0])
        pltpu.sync_copy(rows[0], out_hbm.at[pl.ds(r0, 40)])
    r0 = base + 600
    pltpu.sync_copy(acc.at[pl.ds(r0, 24)], rows[0].at[pl.ds(0, 24)])
    pltpu.sync_copy(rows[0].at[pl.ds(0, 24)], out_hbm.at[pl.ds(r0, 24)])

    @pl.when(s == NS - 1)
    def _():
        r0 = NS * ROWS_PT
        pltpu.sync_copy(acc.at[pl.ds(r0, N - NS * ROWS_PT)],
                        rows[1].at[pl.ds(0, N - NS * ROWS_PT)])
        pltpu.sync_copy(rows[1].at[pl.ds(0, N - NS * ROWS_PT)],
                        out_hbm.at[pl.ds(r0, N - NS * ROWS_PT)])


def _msg_kernel_fn(src, dst4, ew, ht, hl, out_t, out_l,
                   acc, src_flat, dst_loc, ew_flat,
                   rg0, rg1, rg2, rg3,
                   gsem0, gsem1, gsem2, gsem3,
                   ssem0, ssem1, ssem2, ssem3):
    c = lax.axis_index("c")
    rows = (rg0, rg1, rg2, rg3)
    gsem = (gsem0, gsem1, gsem2, gsem3)
    ssem = (ssem0, ssem1, ssem2, ssem3)

    @pl.when(c == 0)
    def _():
        _msg_branch(src, dst4, ew, ht, out_t, acc,
                    src_flat, dst_loc, ew_flat, rows, gsem, ssem)

    @pl.when(c == 1)
    def _():
        _msg_branch(src, dst4, ew, hl, out_l, acc,
                    src_flat, dst_loc, ew_flat, rows, gsem, ssem)


@functools.lru_cache(maxsize=None)
def _msg_call():
    return functools.partial(
    pl.kernel,
    out_type=[jax.ShapeDtypeStruct((N, D), jnp.float32),
              jax.ShapeDtypeStruct((N, D), jnp.float32)],
    mesh=plsc.VectorSubcoreMesh(core_axis_name="c", subcore_axis_name="s"),
    compiler_params=pltpu.CompilerParams(needs_layout_passes=False),
    scratch_types=[
        pltpu.VMEM_SHARED((N, D), jnp.float32),
        pltpu.VMEM((EPP,), jnp.int32),
        pltpu.VMEM((CPP, EK), jnp.int32),
        pltpu.VMEM((EPP,), jnp.float32),
        pltpu.VMEM((EK, D), jnp.float32),
        pltpu.VMEM((EK, D), jnp.float32),
        pltpu.VMEM((EK, D), jnp.float32),
        pltpu.VMEM((EK, D), jnp.float32),
        pltpu.SemaphoreType.DMA,
        pltpu.SemaphoreType.DMA,
        pltpu.SemaphoreType.DMA,
        pltpu.SemaphoreType.DMA,
        pltpu.SemaphoreType.DMA,
        pltpu.SemaphoreType.DMA,
        pltpu.SemaphoreType.DMA,
        pltpu.SemaphoreType.DMA,
    ],
    )(_msg_kernel_fn)


C_BLK = 1000


def _combine_fn(at_ref, wt_ref, bt_ref, al_ref, wl_ref, bl_ref,
                ot_ref, ol_ref):
    xt = jnp.dot(at_ref[...], wt_ref[...],
                 preferred_element_type=jnp.float32) + bt_ref[...]
    ot_ref[...] = jnp.maximum(xt, 0.0)
    xl = jnp.dot(al_ref[...], wl_ref[...],
                 preferred_element_type=jnp.float32) + bl_ref[...]
    ol_ref[...] = jnp.maximum(xl, 0.0)


def _combine(acc_t, wtT, bt, acc_l, wlT, bl):
    blk = lambda i: (i, 0)
    fix = lambda i: (0, 0)
    return pl.pallas_call(
        _combine_fn,
        grid=(N // C_BLK,),
        in_specs=[
            pl.BlockSpec((C_BLK, D), blk),
            pl.BlockSpec((D, H), fix),
            pl.BlockSpec((1, H), fix),
            pl.BlockSpec((C_BLK, D), blk),
            pl.BlockSpec((D, H), fix),
            pl.BlockSpec((1, H), fix),
        ],
        out_specs=[
            pl.BlockSpec((C_BLK, H), blk),
            pl.BlockSpec((C_BLK, H), blk),
        ],
        out_shape=[jax.ShapeDtypeStruct((N, H), jnp.float32),
                   jax.ShapeDtypeStruct((N, H), jnp.float32)],
    )(acc_t, wtT, bt.reshape(1, H), acc_l, wlT, bl.reshape(1, H))


R_BLK = 1000
NR = N // R_BLK


def _readout_fn(ht_ref, hl_ref, g_ref, adap_ref, w1_ref, b1_ref,
                wo_ref, bo_ref, out_ref, acc_t, acc_l, cnt):
    i = pl.program_id(0)

    @pl.when(i == 0)
    def _():
        acc_t[...] = jnp.zeros_like(acc_t)
        acc_l[...] = jnp.zeros_like(acc_l)
        cnt[...] = jnp.zeros_like(cnt)

    g = g_ref[...]
    cols = lax.broadcasted_iota(jnp.int32, (R_BLK, 128), 1)
    onehot = (g == cols).astype(jnp.float32)

    contract = (((0,), (0,)), ((), ()))
    acc_t[...] += lax.dot_general(onehot, ht_ref[...], contract,
                                  preferred_element_type=jnp.float32)
    acc_l[...] += lax.dot_general(onehot, hl_ref[...], contract,
                                  preferred_element_type=jnp.float32)
    ones = jnp.ones((R_BLK, 1), jnp.float32)
    cnt[...] += lax.dot_general(onehot, ones, contract,
                                preferred_element_type=jnp.float32)

    @pl.when(i == NR - 1)
    def _():
        c = jnp.maximum(cnt[...], 1.0)
        r_t = acc_t[...] / c
        r_l = acc_l[...] / c
        a0 = adap_ref[0, 0]
        a1 = adap_ref[0, 1]
        adap_out = a0 * r_t + a1 * r_l
        fused = jnp.maximum(
            jnp.dot(adap_out, w1_ref[...],
                    preferred_element_type=jnp.float32) + b1_ref[...], 0.0)
        res = jnp.dot(fused, wo_ref[...],
                      preferred_element_type=jnp.float32) + bo_ref[...]
        out_ref[...] = res[:B, :]


def _readout(ht, hl, gids2, adap_W, w1T, b1, woT, bo):
    blk = lambda i: (i, 0)
    fix = lambda i: (0, 0)
    return pl.pallas_call(
        _readout_fn,
        grid=(NR,),
        in_specs=[
            pl.BlockSpec((R_BLK, H), blk),
            pl.BlockSpec((R_BLK, H), blk),
            pl.BlockSpec((R_BLK, 1), blk),
            pl.BlockSpec(memory_space=pltpu.SMEM),
            pl.BlockSpec((H, H), fix),
            pl.BlockSpec((1, H), fix),
            pl.BlockSpec((H, OUT), fix),
            pl.BlockSpec((1, OUT), fix),
        ],
        out_specs=pl.BlockSpec((B, OUT), fix),
        out_shape=jax.ShapeDtypeStruct((B, OUT), jnp.float32),
        scratch_shapes=[
            pltpu.VMEM((128, H), jnp.float32),
            pltpu.VMEM((128, H), jnp.float32),
            pltpu.VMEM((128, 1), jnp.float32),
        ],
    )(ht, hl, gids2, adap_W, w1T, b1.reshape(1, H), woT, bo.reshape(1, OUT))


def kernel(edge_index, edge_weight, text_ids, text_lens, label_ids,
           label_lens, graph_ids, embedding_Word, Emb_label,
           W_text_0, b_text_0, W_text_1, b_text_1,
           W_label_0, b_label_0, W_label_1, b_label_1,
           adap_W, fusion_W1, fusion_b1, fusion_Wo, fusion_bo):
    src = edge_index[0].astype(jnp.int32)
    dst = edge_index[1].astype(jnp.int32)
    ew = edge_weight.astype(jnp.float32)

    tl = text_lens.astype(jnp.float32)
    wt = jnp.where(jnp.arange(LT)[None, :] < text_lens[:, None],
                   1.0 / tl[:, None], 0.0).astype(jnp.float32)
    wl = jnp.where(jnp.arange(LL)[None, :] < label_lens[:, None],
                   1.0, 0.0).astype(jnp.float32)

    text_feat, label_feat = _emb_call()(
        text_ids.astype(jnp.int32).reshape(-1), wt.reshape(-1),
        label_ids.astype(jnp.int32).reshape(-1), wl.reshape(-1),
        embedding_Word, Emb_label)

    dst4 = dst.reshape(NS, NPH, CPP, EK)
    acc_t, acc_l = _msg_call()(src, dst4, ew, text_feat, label_feat)
    h1t, h1l = _combine(acc_t, W_text_0.T, b_text_0,
                        acc_l, W_label_0.T, b_label_0)
    acc_t2, acc_l2 = _msg_call()(src, dst4, ew, h1t, h1l)
    h2t, h2l = _combine(acc_t2, W_text_1.T, b_text_1,
                        acc_l2, W_label_1.T, b_label_1)

    res = _readout(h2t, h2l, graph_ids.astype(jnp.int32).reshape(N, 1),
                   adap_W, fusion_W1.T, fusion_b1, fusion_Wo.T, fusion_bo)
    return res


# speedup vs baseline: 7.5399x; 1.0512x over previous
import functools

import jax
import jax.numpy as jnp
from jax import lax
from jax.experimental import pallas as pl
from jax.experimental.pallas import tpu as pltpu
from jax.experimental.pallas import tpu_sc as plsc

N = 10000
E = 320000
B = 100
D = 128
H = 128
OUT = 256
LT = 50
LL = 10

NC = 2
NS = 16
NW = NC * NS

NODE_BLK = 40
NUM_BLKS = N // NODE_BLK
EPAD = 327680
EPT = EPAD // NS
EK = 64
NCHUNK = EPT // EK
ROWS_PT = 624
INIT_CH = 208


def _full16(v):
    return jnp.full((16,), v, jnp.int32)


BN = 8
NBT = NODE_BLK // BN


def _emb_body(ids_hbm, w_hbm, table_hbm, out_hbm, ids_v, w_v, rows, obuf,
              sems, nwords):
    c = lax.axis_index("c")
    s = lax.axis_index("s")
    w = s * NC + c

    nblk = 7 + jnp.where(w < NUM_BLKS - 7 * NW, 1, 0)
    bw = BN * nwords

    def issue_gather(bt, par):
        pltpu.async_copy(table_hbm.at[ids_v.at[pl.ds(bt * bw, bw)]],
                         rows[par].at[pl.ds(0, bw)], sems[par])

    def block_body(i, _):
        blk = w + i * NW
        nb0 = blk * NODE_BLK
        pltpu.sync_copy(ids_hbm.at[pl.ds(nb0 * nwords, NODE_BLK * nwords)],
                        ids_v)
        pltpu.sync_copy(w_hbm.at[pl.ds(nb0 * nwords, NODE_BLK * nwords)], w_v)

        issue_gather(0, 0)

        def batch_body(bt, par):
            pltpu.make_async_copy(table_hbm.at[ids_v.at[pl.ds(bt * bw, bw)]],
                                  rows[par].at[pl.ds(0, bw)],
                                  sems[par]).wait()

            @pl.when(bt + 1 < NBT)
            def _():
                issue_gather(bt + 1, 1 - par)

            def node_body(u, _):
                wbase = (bt * BN + u) * nwords
                rbase = u * nwords
                wj = plsc.load_gather(w_v, [_full16(wbase)])
                acc = [rows[par][rbase, pl.ds(l * 16, 16)] * wj
                       for l in range(8)]
                for j in range(1, nwords):
                    wj = plsc.load_gather(w_v, [_full16(wbase + j)])
                    for l in range(8):
                        acc[l] = (acc[l]
                                  + rows[par][rbase + j, pl.ds(l * 16, 16)]
                                  * wj)
                for l in range(8):
                    obuf[u, pl.ds(l * 16, 16)] = acc[l]
                return _

            lax.fori_loop(0, BN, node_body, 0)
            pltpu.sync_copy(obuf, out_hbm.at[pl.ds(nb0 + bt * BN, BN)])

        def pair_body(p, _):
            batch_body(p * 2, 0)
            batch_body(p * 2 + 1, 1)
            return _

        lax.fori_loop(0, NBT // 2, pair_body, 0)
        if NBT % 2:
            batch_body(NBT - 1, 0)
        return _

    lax.fori_loop(0, nblk, block_body, 0)


def _emb_kernel_fn(text_ids, wt, label_ids, wl, word_tab, label_tab,
                   out_t, out_l,
                   ids_tv, wt_v, rt0, rt1, ids_lv, wl_v, obuf,
                   sem0, sem1):
    _emb_body(text_ids, wt, word_tab, out_t, ids_tv, wt_v, (rt0, rt1), obuf,
              (sem0, sem1), LT)
    _emb_body(label_ids, wl, label_tab, out_l, ids_lv, wl_v, (rt0, rt1), obuf,
              (sem0, sem1), LL)


@functools.lru_cache(maxsize=None)
def _emb_call():
    return functools.partial(
    pl.kernel,
    out_type=[jax.ShapeDtypeStruct((N, D), jnp.float32),
              jax.ShapeDtypeStruct((N, D), jnp.float32)],
    mesh=plsc.VectorSubcoreMesh(core_axis_name="c", subcore_axis_name="s"),
    compiler_params=pltpu.CompilerParams(needs_layout_passes=False),
    scratch_types=[
        pltpu.VMEM((NODE_BLK * LT,), jnp.int32),
        pltpu.VMEM((NODE_BLK * LT,), jnp.float32),
        pltpu.VMEM((BN * LT, D), jnp.float32),
        pltpu.VMEM((BN * LT, D), jnp.float32),
        pltpu.VMEM((NODE_BLK * LL,), jnp.int32),
        pltpu.VMEM((NODE_BLK * LL,), jnp.float32),
        pltpu.VMEM((BN, D), jnp.float32),
        pltpu.SemaphoreType.DMA,
        pltpu.SemaphoreType.DMA,
    ],
    )(_emb_kernel_fn)


NPH = 5
EPP = EPT // NPH
CPP = NCHUNK // NPH
NBUF = 4


def _msg_branch(src_hbm, dst4_hbm, ew_hbm, h_hbm, out_hbm, acc,
                src_flat, dst_loc, ew_flat, rows, gsem, ssem):
    s = lax.axis_index("s")

    base = s * ROWS_PT
    for ci in range(13):
        r0 = base + ci * 48
        pltpu.sync_copy(h_hbm.at[pl.ds(r0, 48)], rows[0].at[pl.ds(0, 48)])
        pltpu.sync_copy(rows[0].at[pl.ds(0, 48)], acc.at[pl.ds(r0, 48)])

    @pl.when(s == NS - 1)
    def _():
        r0 = NS * ROWS_PT
        pltpu.sync_copy(h_hbm.at[pl.ds(r0, N - NS * ROWS_PT)],
                        rows[1].at[pl.ds(0, N - NS * ROWS_PT)])
        pltpu.sync_copy(rows[1].at[pl.ds(0, N - NS * ROWS_PT)],
                        acc.at[pl.ds(r0, N - NS * ROWS_PT)])

    plsc.subcore_barrier()

    dst_s = dst4_hbm.at[s]

    def issue_gather(m, r):
        pltpu.async_copy(h_hbm.at[src_flat.at[pl.ds(m * EK, EK)]],
                         rows[r], gsem[r])

    def wait_gather(m, r):
        pltpu.make_async_copy(h_hbm.at[src_flat.at[pl.ds(m * EK, EK)]],
                              rows[r], gsem[r]).wait()

    def wait_scatter(m, r):
        pltpu.make_async_copy(rows[r], acc.at[dst_loc.at[m]], ssem[r]).wait()

    def chunk_body(m, r):
        @pl.when(m >= 2)
        def _():
            wait_scatter(m - 2, r ^ 2)

        @pl.when(m + 2 < CPP)
        def _():
            issue_gather(m + 2, r ^ 2)

        wait_gather(m, r)
        wb = m * EK

        def scale2(e2, _):
            e = e2 * 2
            for u in range(2):
                wv = plsc.load_gather(
                    ew_flat, [jnp.full((16,), wb + e + u, jnp.int32)])
                for l in range(8):
                    rows[r][e + u, pl.ds(l * 16, 16)] = (
                        rows[r][e + u, pl.ds(l * 16, 16)] * wv)
            return _

        lax.fori_loop(0, EK // 2, scale2, 0)

        pltpu.async_copy(rows[r], acc.at[dst_loc.at[m]], ssem[r], add=True)

    def phase_body(ph, _):
        e0 = s * EPT + ph * EPP
        pltpu.sync_copy(src_hbm.at[pl.ds(e0, EPP)], src_flat)
        pltpu.sync_copy(ew_hbm.at[pl.ds(e0, EPP)], ew_flat)
        pltpu.sync_copy(dst_s.at[ph], dst_loc)

        issue_gather(0, 0)
        issue_gather(1, 1)

        def iter_body(it, _):
            for u in range(NBUF):
                chunk_body(it * NBUF + u, u)
            return _

        lax.fori_loop(0, CPP // NBUF, iter_body, 0)
        wait_scatter(CPP - 2, (CPP - 2) % NBUF)
        wait_scatter(CPP - 1, (CPP - 1) % NBUF)
        return _

    lax.fori_loop(0, NPH, phase_body, 0)

    plsc.subcore_barrier()

    for ci in range(13):
        r0 = base + ci * 48
        pltpu.sync_copy(acc.at[pl.ds(r0, 48)], rows[0].at[pl.ds(0, 48)])
        pltpu.sync_copy(rows[0].at[pl.ds(0, 48)], out_hbm.at[pl.ds(r0, 48)])

    @pl.when(s == NS - 1)
    def _():
        r0 = NS * ROWS_PT
        pltpu.sync_copy(acc.at[pl.ds(r0, N - NS * ROWS_PT)],
                        rows[1].at[pl.ds(0, N - NS * ROWS_PT)])
        pltpu.sync_copy(rows[1].at[pl.ds(0, N - NS * ROWS_PT)],
                        out_hbm.at[pl.ds(r0, N - NS * ROWS_PT)])


def _msg_kernel_fn(src, dst4, ew, ht, hl, out_t, out_l,
                   acc, src_flat, dst_loc, ew_flat,
                   rg0, rg1, rg2, rg3,
                   gsem0, gsem1, gsem2, gsem3,
                   ssem0, ssem1, ssem2, ssem3):
    c = lax.axis_index("c")
    rows = (rg0, rg1, rg2, rg3)
    gsem = (gsem0, gsem1, gsem2, gsem3)
    ssem = (ssem0, ssem1, ssem2, ssem3)

    @pl.when(c == 0)
    def _():
        _msg_branch(src, dst4, ew, ht, out_t, acc,
                    src_flat, dst_loc, ew_flat, rows, gsem, ssem)

    @pl.when(c == 1)
    def _():
        _msg_branch(src, dst4, ew, hl, out_l, acc,
                    src_flat, dst_loc, ew_flat, rows, gsem, ssem)


@functools.lru_cache(maxsize=None)
def _msg_call():
    return functools.partial(
    pl.kernel,
    out_type=[jax.ShapeDtypeStruct((N, D), jnp.float32),
              jax.ShapeDtypeStruct((N, D), jnp.float32)],
    mesh=plsc.VectorSubcoreMesh(core_axis_name="c", subcore_axis_name="s"),
    compiler_params=pltpu.CompilerParams(needs_layout_passes=False),
    scratch_types=[
        pltpu.VMEM_SHARED((N, D), jnp.float32),
        pltpu.VMEM((EPP,), jnp.int32),
        pltpu.VMEM((CPP, EK), jnp.int32),
        pltpu.VMEM((EPP,), jnp.float32),
        pltpu.VMEM((EK, D), jnp.float32),
        pltpu.VMEM((EK, D), jnp.float32),
        pltpu.VMEM((EK, D), jnp.float32),
        pltpu.VMEM((EK, D), jnp.float32),
        pltpu.SemaphoreType.DMA,
        pltpu.SemaphoreType.DMA,
        pltpu.SemaphoreType.DMA,
        pltpu.SemaphoreType.DMA,
        pltpu.SemaphoreType.DMA,
        pltpu.SemaphoreType.DMA,
        pltpu.SemaphoreType.DMA,
        pltpu.SemaphoreType.DMA,
    ],
    )(_msg_kernel_fn)


C_BLK = 1000


def _combine_fn(at_ref, wt_ref, bt_ref, al_ref, wl_ref, bl_ref,
                ot_ref, ol_ref):
    xt = jnp.dot(at_ref[...], wt_ref[...],
                 preferred_element_type=jnp.float32) + bt_ref[...]
    ot_ref[...] = jnp.maximum(xt, 0.0)
    xl = jnp.dot(al_ref[...], wl_ref[...],
                 preferred_element_type=jnp.float32) + bl_ref[...]
    ol_ref[...] = jnp.maximum(xl, 0.0)


def _combine(acc_t, wtT, bt, acc_l, wlT, bl):
    blk = lambda i: (i, 0)
    fix = lambda i: (0, 0)
    return pl.pallas_call(
        _combine_fn,
        grid=(N // C_BLK,),
        in_specs=[
            pl.BlockSpec((C_BLK, D), blk),
            pl.BlockSpec((D, H), fix),
            pl.BlockSpec((1, H), fix),
            pl.BlockSpec((C_BLK, D), blk),
            pl.BlockSpec((D, H), fix),
            pl.BlockSpec((1, H), fix),
        ],
        out_specs=[
            pl.BlockSpec((C_BLK, H), blk),
            pl.BlockSpec((C_BLK, H), blk),
        ],
        out_shape=[jax.ShapeDtypeStruct((N, H), jnp.float32),
                   jax.ShapeDtypeStruct((N, H), jnp.float32)],
    )(acc_t, wtT, bt.reshape(1, H), acc_l, wlT, bl.reshape(1, H))


R_BLK = 1000
NR = N // R_BLK


def _readout_fn(ht_ref, hl_ref, g_ref, adap_ref, w1_ref, b1_ref,
                wo_ref, bo_ref, out_ref, acc_t, acc_l, cnt):
    i = pl.program_id(0)

    @pl.when(i == 0)
    def _():
        acc_t[...] = jnp.zeros_like(acc_t)
        acc_l[...] = jnp.zeros_like(acc_l)
        cnt[...] = jnp.zeros_like(cnt)

    g = g_ref[...]
    cols = lax.broadcasted_iota(jnp.int32, (R_BLK, 128), 1)
    onehot = (g == cols).astype(jnp.float32)

    contract = (((0,), (0,)), ((), ()))
    acc_t[...] += lax.dot_general(onehot, ht_ref[...], contract,
                                  preferred_element_type=jnp.float32)
    acc_l[...] += lax.dot_general(onehot, hl_ref[...], contract,
                                  preferred_element_type=jnp.float32)
    ones = jnp.ones((R_BLK, 1), jnp.float32)
    cnt[...] += lax.dot_general(onehot, ones, contract,
                                preferred_element_type=jnp.float32)

    @pl.when(i == NR - 1)
    def _():
        c = jnp.maximum(cnt[...], 1.0)
        r_t = acc_t[...] / c
        r_l = acc_l[...] / c
        a0 = adap_ref[0, 0]
        a1 = adap_ref[0, 1]
        adap_out = a0 * r_t + a1 * r_l
        fused = jnp.maximum(
            jnp.dot(adap_out, w1_ref[...],
                    preferred_element_type=jnp.float32) + b1_ref[...], 0.0)
        res = jnp.dot(fused, wo_ref[...],
                      preferred_element_type=jnp.float32) + bo_ref[...]
        out_ref[...] = res[:B, :]


def _readout(ht, hl, gids2, adap_W, w1T, b1, woT, bo):
    blk = lambda i: (i, 0)
    fix = lambda i: (0, 0)
    return pl.pallas_call(
        _readout_fn,
        grid=(NR,),
        in_specs=[
            pl.BlockSpec((R_BLK, H), blk),
            pl.BlockSpec((R_BLK, H), blk),
            pl.BlockSpec((R_BLK, 1), blk),
            pl.BlockSpec(memory_space=pltpu.SMEM),
            pl.BlockSpec((H, H), fix),
            pl.BlockSpec((1, H), fix),
            pl.BlockSpec((H, OUT), fix),
            pl.BlockSpec((1, OUT), fix),
        ],
        out_specs=pl.BlockSpec((B, OUT), fix),
        out_shape=jax.ShapeDtypeStruct((B, OUT), jnp.float32),
        scratch_shapes=[
            pltpu.VMEM((128, H), jnp.float32),
            pltpu.VMEM((128, H), jnp.float32),
            pltpu.VMEM((128, 1), jnp.float32),
        ],
    )(ht, hl, gids2, adap_W, w1T, b1.reshape(1, H), woT, bo.reshape(1, OUT))


def kernel(edge_index, edge_weight, text_ids, text_lens, label_ids,
           label_lens, graph_ids, embedding_Word, Emb_label,
           W_text_0, b_text_0, W_text_1, b_text_1,
           W_label_0, b_label_0, W_label_1, b_label_1,
           adap_W, fusion_W1, fusion_b1, fusion_Wo, fusion_bo):
    pad_idx = (jnp.arange(EPAD - E, dtype=jnp.int32) % N)
    src = jnp.concatenate([edge_index[0].astype(jnp.int32), pad_idx])
    dst = jnp.concatenate([edge_index[1].astype(jnp.int32), pad_idx])
    ew = jnp.concatenate([edge_weight.astype(jnp.float32),
                          jnp.zeros((EPAD - E,), jnp.float32)])

    tl = text_lens.astype(jnp.float32)
    wt = jnp.where(jnp.arange(LT)[None, :] < text_lens[:, None],
                   1.0 / tl[:, None], 0.0).astype(jnp.float32)
    wl = jnp.where(jnp.arange(LL)[None, :] < label_lens[:, None],
                   1.0, 0.0).astype(jnp.float32)

    text_feat, label_feat = _emb_call()(
        text_ids.astype(jnp.int32).reshape(-1), wt.reshape(-1),
        label_ids.astype(jnp.int32).reshape(-1), wl.reshape(-1),
        embedding_Word, Emb_label)

    dst4 = dst.reshape(NS, NPH, CPP, EK)
    acc_t, acc_l = _msg_call()(src, dst4, ew, text_feat, label_feat)
    h1t, h1l = _combine(acc_t, W_text_0.T, b_text_0,
                        acc_l, W_label_0.T, b_label_0)
    acc_t2, acc_l2 = _msg_call()(src, dst4, ew, h1t, h1l)
    h2t, h2l = _combine(acc_t2, W_text_1.T, b_text_1,
                        acc_l2, W_label_1.T, b_label_1)

    res = _readout(h2t, h2l, graph_ids.astype(jnp.int32).reshape(N, 1),
                   adap_W, fusion_W1.T, fusion_b1, fusion_Wo.T, fusion_bo)
    return res


# speedup vs baseline: 7.6213x; 1.0108x over previous
import functools

import jax
import jax.numpy as jnp
from jax import lax
from jax.experimental import pallas as pl
from jax.experimental.pallas import tpu as pltpu
from jax.experimental.pallas import tpu_sc as plsc

N = 10000
E = 320000
B = 100
D = 128
H = 128
OUT = 256
LT = 50
LL = 10

NC = 2
NS = 16
NW = NC * NS

NODE_BLK = 40
NUM_BLKS = N // NODE_BLK
EPAD = 327680
EPT = EPAD // NS
EK = 64
NCHUNK = EPT // EK
ROWS_PT = 624
INIT_CH = 208


def _full16(v):
    return jnp.full((16,), v, jnp.int32)


BN = 8
NBT = NODE_BLK // BN


def _emb_body(ids_hbm, w_hbm, table_hbm, out_hbm, ids_v, w_v, rows, obuf,
              sems, nwords):
    c = lax.axis_index("c")
    s = lax.axis_index("s")
    w = s * NC + c

    nblk = 7 + jnp.where(w < NUM_BLKS - 7 * NW, 1, 0)
    bw = BN * nwords

    def issue_gather(bt, par):
        pltpu.async_copy(table_hbm.at[ids_v.at[pl.ds(bt * bw, bw)]],
                         rows[par].at[pl.ds(0, bw)], sems[par])

    def block_body(i, _):
        blk = w + i * NW
        nb0 = blk * NODE_BLK
        pltpu.sync_copy(ids_hbm.at[pl.ds(nb0 * nwords, NODE_BLK * nwords)],
                        ids_v)
        pltpu.sync_copy(w_hbm.at[pl.ds(nb0 * nwords, NODE_BLK * nwords)], w_v)

        issue_gather(0, 0)

        def batch_body(bt, par):
            pltpu.make_async_copy(table_hbm.at[ids_v.at[pl.ds(bt * bw, bw)]],
                                  rows[par].at[pl.ds(0, bw)],
                                  sems[par]).wait()

            @pl.when(bt + 1 < NBT)
            def _():
                issue_gather(bt + 1, 1 - par)

            def node_body(u, _):
                wbase = (bt * BN + u) * nwords
                rbase = u * nwords
                wj = plsc.load_gather(w_v, [_full16(wbase)])
                acc = [rows[par][rbase, pl.ds(l * 16, 16)] * wj
                       for l in range(8)]
                for j in range(1, nwords):
                    wj = plsc.load_gather(w_v, [_full16(wbase + j)])
                    for l in range(8):
                        acc[l] = (acc[l]
                                  + rows[par][rbase + j, pl.ds(l * 16, 16)]
                                  * wj)
                for l in range(8):
                    obuf[u, pl.ds(l * 16, 16)] = acc[l]
                return _

            lax.fori_loop(0, BN, node_body, 0)
            pltpu.sync_copy(obuf, out_hbm.at[pl.ds(nb0 + bt * BN, BN)])

        def pair_body(p, _):
            batch_body(p * 2, 0)
            batch_body(p * 2 + 1, 1)
            return _

        lax.fori_loop(0, NBT // 2, pair_body, 0)
        if NBT % 2:
            batch_body(NBT - 1, 0)
        return _

    lax.fori_loop(0, nblk, block_body, 0)


def _emb_kernel_fn(text_ids, wt, label_ids, wl, word_tab, label_tab,
                   out_t, out_l,
                   ids_tv, wt_v, rt0, rt1, ids_lv, wl_v, obuf,
                   sem0, sem1):
    _emb_body(text_ids, wt, word_tab, out_t, ids_tv, wt_v, (rt0, rt1), obuf,
              (sem0, sem1), LT)
    _emb_body(label_ids, wl, label_tab, out_l, ids_lv, wl_v, (rt0, rt1), obuf,
              (sem0, sem1), LL)


@functools.lru_cache(maxsize=None)
def _emb_call():
    return functools.partial(
    pl.kernel,
    out_type=[jax.ShapeDtypeStruct((N, D), jnp.float32),
              jax.ShapeDtypeStruct((N, D), jnp.float32)],
    mesh=plsc.VectorSubcoreMesh(core_axis_name="c", subcore_axis_name="s"),
    compiler_params=pltpu.CompilerParams(needs_layout_passes=False),
    scratch_types=[
        pltpu.VMEM((NODE_BLK * LT,), jnp.int32),
        pltpu.VMEM((NODE_BLK * LT,), jnp.float32),
        pltpu.VMEM((BN * LT, D), jnp.float32),
        pltpu.VMEM((BN * LT, D), jnp.float32),
        pltpu.VMEM((NODE_BLK * LL,), jnp.int32),
        pltpu.VMEM((NODE_BLK * LL,), jnp.float32),
        pltpu.VMEM((BN, D), jnp.float32),
        pltpu.SemaphoreType.DMA,
        pltpu.SemaphoreType.DMA,
    ],
    )(_emb_kernel_fn)


NPH = 5
EPP = EPT // NPH
CPP = NCHUNK // NPH
NBUF = 4


def _msg_branch(src_hbm, dst4_hbm, ew_hbm, h_hbm, out_hbm, acc,
                src_flat, dst_loc, ew_flat, rows, gsem, ssem):
    s = lax.axis_index("s")

    base = s * ROWS_PT
    for ci in range(13):
        r0 = base + ci * 48
        pltpu.sync_copy(h_hbm.at[pl.ds(r0, 48)], rows[0].at[pl.ds(0, 48)])
        pltpu.sync_copy(rows[0].at[pl.ds(0, 48)], acc.at[pl.ds(r0, 48)])

    @pl.when(s == NS - 1)
    def _():
        r0 = NS * ROWS_PT
        pltpu.sync_copy(h_hbm.at[pl.ds(r0, N - NS * ROWS_PT)],
                        rows[1].at[pl.ds(0, N - NS * ROWS_PT)])
        pltpu.sync_copy(rows[1].at[pl.ds(0, N - NS * ROWS_PT)],
                        acc.at[pl.ds(r0, N - NS * ROWS_PT)])

    plsc.subcore_barrier()

    dst_s = dst4_hbm.at[s]

    def issue_gather(m, r):
        pltpu.async_copy(h_hbm.at[src_flat.at[pl.ds(m * EK, EK)]],
                         rows[r], gsem[r])

    def wait_gather(m, r):
        pltpu.make_async_copy(h_hbm.at[src_flat.at[pl.ds(m * EK, EK)]],
                              rows[r], gsem[r]).wait()

    def wait_scatter(m, r):
        pltpu.make_async_copy(rows[r], acc.at[dst_loc.at[m]], ssem[r]).wait()

    def chunk_body(m, r):
        @pl.when(m >= 2)
        def _():
            wait_scatter(m - 2, r ^ 2)

        @pl.when(m + 2 < CPP)
        def _():
            issue_gather(m + 2, r ^ 2)

        wait_gather(m, r)
        wb = m * EK

        def scale2(e2, _):
            e = e2 * 4
            for u in range(4):
                wv = plsc.load_gather(
                    ew_flat, [jnp.full((16,), wb + e + u, jnp.int32)])
                for l in range(8):
                    rows[r][e + u, pl.ds(l * 16, 16)] = (
                        rows[r][e + u, pl.ds(l * 16, 16)] * wv)
            return _

        lax.fori_loop(0, EK // 4, scale2, 0)

        pltpu.async_copy(rows[r], acc.at[dst_loc.at[m]], ssem[r], add=True)

    def phase_body(ph, _):
        e0 = s * EPT + ph * EPP
        pltpu.sync_copy(src_hbm.at[pl.ds(e0, EPP)], src_flat)
        pltpu.sync_copy(ew_hbm.at[pl.ds(e0, EPP)], ew_flat)
        pltpu.sync_copy(dst_s.at[ph], dst_loc)

        issue_gather(0, 0)
        issue_gather(1, 1)

        def iter_body(it, _):
            for u in range(NBUF):
                chunk_body(it * NBUF + u, u)
            return _

        lax.fori_loop(0, CPP // NBUF, iter_body, 0)
        wait_scatter(CPP - 2, (CPP - 2) % NBUF)
        wait_scatter(CPP - 1, (CPP - 1) % NBUF)
        return _

    lax.fori_loop(0, NPH, phase_body, 0)

    plsc.subcore_barrier()

    for ci in range(13):
        r0 = base + ci * 48
        pltpu.sync_copy(acc.at[pl.ds(r0, 48)], rows[0].at[pl.ds(0, 48)])
        pltpu.sync_copy(rows[0].at[pl.ds(0, 48)], out_hbm.at[pl.ds(r0, 48)])

    @pl.when(s == NS - 1)
    def _():
        r0 = NS * ROWS_PT
        pltpu.sync_copy(acc.at[pl.ds(r0, N - NS * ROWS_PT)],
                        rows[1].at[pl.ds(0, N - NS * ROWS_PT)])
        pltpu.sync_copy(rows[1].at[pl.ds(0, N - NS * ROWS_PT)],
                        out_hbm.at[pl.ds(r0, N - NS * ROWS_PT)])


def _msg_kernel_fn(src, dst4, ew, ht, hl, out_t, out_l,
                   acc, src_flat, dst_loc, ew_flat,
                   rg0, rg1, rg2, rg3,
                   gsem0, gsem1, gsem2, gsem3,
                   ssem0, ssem1, ssem2, ssem3):
    c = lax.axis_index("c")
    rows = (rg0, rg1, rg2, rg3)
    gsem = (gsem0, gsem1, gsem2, gsem3)
    ssem = (ssem0, ssem1, ssem2, ssem3)

    @pl.when(c == 0)
    def _():
        _msg_branch(src, dst4, ew, ht, out_t, acc,
                    src_flat, dst_loc, ew_flat, rows, gsem, ssem)

    @pl.when(c == 1)
    def _():
        _msg_branch(src, dst4, ew, hl, out_l, acc,
                    src_flat, dst_loc, ew_flat, rows, gsem, ssem)


@functools.lru_cache(maxsize=None)
def _msg_call():
    return functools.partial(
    pl.kernel,
    out_type=[jax.ShapeDtypeStruct((N, D), jnp.float32),
              jax.ShapeDtypeStruct((N, D), jnp.float32)],
    mesh=plsc.VectorSubcoreMesh(core_axis_name="c", subcore_axis_name="s"),
    compiler_params=pltpu.CompilerParams(needs_layout_passes=False),
    scratch_types=[
        pltpu.VMEM_SHARED((N, D), jnp.float32),
        pltpu.VMEM((EPP,), jnp.int32),
        pltpu.VMEM((CPP, EK), jnp.int32),
        pltpu.VMEM((EPP,), jnp.float32),
        pltpu.VMEM((EK, D), jnp.float32),
        pltpu.VMEM((EK, D), jnp.float32),
        pltpu.VMEM((EK, D), jnp.float32),
        pltpu.VMEM((EK, D), jnp.float32),
        pltpu.SemaphoreType.DMA,
        pltpu.SemaphoreType.DMA,
        pltpu.SemaphoreType.DMA,
        pltpu.SemaphoreType.DMA,
        pltpu.SemaphoreType.DMA,
        pltpu.SemaphoreType.DMA,
        pltpu.SemaphoreType.DMA,
        pltpu.SemaphoreType.DMA,
    ],
    )(_msg_kernel_fn)


C_BLK = 1000


def _combine_fn(at_ref, wt_ref, bt_ref, al_ref, wl_ref, bl_ref,
                ot_ref, ol_ref):
    xt = jnp.dot(at_ref[...], wt_ref[...],
                 preferred_element_type=jnp.float32) + bt_ref[...]
    ot_ref[...] = jnp.maximum(xt, 0.0)
    xl = jnp.dot(al_ref[...], wl_ref[...],
                 preferred_element_type=jnp.float32) + bl_ref[...]
    ol_ref[...] = jnp.maximum(xl, 0.0)


def _combine(acc_t, wtT, bt, acc_l, wlT, bl):
    blk = lambda i: (i, 0)
    fix = lambda i: (0, 0)
    return pl.pallas_call(
        _combine_fn,
        grid=(N // C_BLK,),
        in_specs=[
            pl.BlockSpec((C_BLK, D), blk),
            pl.BlockSpec((D, H), fix),
            pl.BlockSpec((1, H), fix),
            pl.BlockSpec((C_BLK, D), blk),
            pl.BlockSpec((D, H), fix),
            pl.BlockSpec((1, H), fix),
        ],
        out_specs=[
            pl.BlockSpec((C_BLK, H), blk),
            pl.BlockSpec((C_BLK, H), blk),
        ],
        out_shape=[jax.ShapeDtypeStruct((N, H), jnp.float32),
                   jax.ShapeDtypeStruct((N, H), jnp.float32)],
    )(acc_t, wtT, bt.reshape(1, H), acc_l, wlT, bl.reshape(1, H))


R_BLK = 1000
NR = N // R_BLK


def _readout_fn(at_ref, wt_ref, bt_ref, al_ref, wl_ref, bl_ref,
                g_ref, adap_ref, w1_ref, b1_ref,
                wo_ref, bo_ref, out_ref, acc_t, acc_l, cnt):
    i = pl.program_id(0)

    @pl.when(i == 0)
    def _():
        acc_t[...] = jnp.zeros_like(acc_t)
        acc_l[...] = jnp.zeros_like(acc_l)
        cnt[...] = jnp.zeros_like(cnt)

    ht = jnp.maximum(jnp.dot(at_ref[...], wt_ref[...],
                             preferred_element_type=jnp.float32)
                     + bt_ref[...], 0.0)
    hl = jnp.maximum(jnp.dot(al_ref[...], wl_ref[...],
                             preferred_element_type=jnp.float32)
                     + bl_ref[...], 0.0)

    g = g_ref[...]
    cols = lax.broadcasted_iota(jnp.int32, (R_BLK, 128), 1)
    onehot = (g == cols).astype(jnp.float32)

    contract = (((0,), (0,)), ((), ()))
    acc_t[...] += lax.dot_general(onehot, ht, contract,
                                  preferred_element_type=jnp.float32)
    acc_l[...] += lax.dot_general(onehot, hl, contract,
                                  preferred_element_type=jnp.float32)
    ones = jnp.ones((R_BLK, 1), jnp.float32)
    cnt[...] += lax.dot_general(onehot, ones, contract,
                                preferred_element_type=jnp.float32)

    @pl.when(i == NR - 1)
    def _():
        c = jnp.maximum(cnt[...], 1.0)
        r_t = acc_t[...] / c
        r_l = acc_l[...] / c
        a0 = adap_ref[0, 0]
        a1 = adap_ref[0, 1]
        adap_out = a0 * r_t + a1 * r_l
        fused = jnp.maximum(
            jnp.dot(adap_out, w1_ref[...],
                    preferred_element_type=jnp.float32) + b1_ref[...], 0.0)
        res = jnp.dot(fused, wo_ref[...],
                      preferred_element_type=jnp.float32) + bo_ref[...]
        out_ref[...] = res[:B, :]


def _readout(at2, wtT, bt, al2, wlT, bl, gids2, adap_W, w1T, b1, woT, bo):
    blk = lambda i: (i, 0)
    fix = lambda i: (0, 0)
    return pl.pallas_call(
        _readout_fn,
        grid=(NR,),
        in_specs=[
            pl.BlockSpec((R_BLK, H), blk),
            pl.BlockSpec((D, H), fix),
            pl.BlockSpec((1, H), fix),
            pl.BlockSpec((R_BLK, H), blk),
            pl.BlockSpec((D, H), fix),
            pl.BlockSpec((1, H), fix),
            pl.BlockSpec((R_BLK, 1), blk),
            pl.BlockSpec(memory_space=pltpu.SMEM),
            pl.BlockSpec((H, H), fix),
            pl.BlockSpec((1, H), fix),
            pl.BlockSpec((H, OUT), fix),
            pl.BlockSpec((1, OUT), fix),
        ],
        out_specs=pl.BlockSpec((B, OUT), fix),
        out_shape=jax.ShapeDtypeStruct((B, OUT), jnp.float32),
        scratch_shapes=[
            pltpu.VMEM((128, H), jnp.float32),
            pltpu.VMEM((128, H), jnp.float32),
            pltpu.VMEM((128, 1), jnp.float32),
        ],
    )(at2, wtT, bt.reshape(1, H), al2, wlT, bl.reshape(1, H), gids2,
      adap_W, w1T, b1.reshape(1, H), woT, bo.reshape(1, OUT))


def kernel(edge_index, edge_weight, text_ids, text_lens, label_ids,
           label_lens, graph_ids, embedding_Word, Emb_label,
           W_text_0, b_text_0, W_text_1, b_text_1,
           W_label_0, b_label_0, W_label_1, b_label_1,
           adap_W, fusion_W1, fusion_b1, fusion_Wo, fusion_bo):
    pad_idx = (jnp.arange(EPAD - E, dtype=jnp.int32) % N)
    src = jnp.concatenate([edge_index[0].astype(jnp.int32), pad_idx])
    dst = jnp.concatenate([edge_index[1].astype(jnp.int32), pad_idx])
    ew = jnp.concatenate([edge_weight.astype(jnp.float32),
                          jnp.zeros((EPAD - E,), jnp.float32)])

    tl = text_lens.astype(jnp.float32)
    wt = jnp.where(jnp.arange(LT)[None, :] < text_lens[:, None],
                   1.0 / tl[:, None], 0.0).astype(jnp.float32)
    wl = jnp.where(jnp.arange(LL)[None, :] < label_lens[:, None],
                   1.0, 0.0).astype(jnp.float32)

    text_feat, label_feat = _emb_call()(
        text_ids.astype(jnp.int32).reshape(-1), wt.reshape(-1),
        label_ids.astype(jnp.int32).reshape(-1), wl.reshape(-1),
        embedding_Word, Emb_label)

    dst4 = dst.reshape(NS, NPH, CPP, EK)
    acc_t, acc_l = _msg_call()(src, dst4, ew, text_feat, label_feat)
    h1t, h1l = _combine(acc_t, W_text_0.T, b_text_0,
                        acc_l, W_label_0.T, b_label_0)
    acc_t2, acc_l2 = _msg_call()(src, dst4, ew, h1t, h1l)

    res = _readout(acc_t2, W_text_1.T, b_text_1,
                   acc_l2, W_label_1.T, b_label_1,
                   graph_ids.astype(jnp.int32).reshape(N, 1),
                   adap_W, fusion_W1.T, fusion_b1, fusion_Wo.T, fusion_bo)
    return res


# speedup vs baseline: 7.6435x; 1.0029x over previous
import functools

import jax
import jax.numpy as jnp
from jax import lax
from jax.experimental import pallas as pl
from jax.experimental.pallas import tpu as pltpu
from jax.experimental.pallas import tpu_sc as plsc

N = 10000
E = 320000
B = 100
D = 128
H = 128
OUT = 256
LT = 50
LL = 10

NC = 2
NS = 16
NW = NC * NS

NODE_BLK = 40
NUM_BLKS = N // NODE_BLK
EPAD = 327680
EPT = EPAD // NS
EK = 64
NCHUNK = EPT // EK
ROWS_PT = 624
INIT_CH = 208


def _full16(v):
    return jnp.full((16,), v, jnp.int32)


BN = 8
NBT = NODE_BLK // BN


def _emb_body(ids_hbm, w_hbm, table_hbm, out_hbm, ids_v, w_v, rows, obuf,
              sems, nwords, packed=False):
    c = lax.axis_index("c")
    s = lax.axis_index("s")
    w = s * NC + c

    nblk = 7 + jnp.where(w < NUM_BLKS - 7 * NW, 1, 0)
    bw = BN * nwords

    def issue_gather(bt, par):
        pltpu.async_copy(table_hbm.at[ids_v.at[pl.ds(bt * bw, bw)]],
                         rows[par].at[pl.ds(0, bw)], sems[par])

    def block_body(i, _):
        blk = w + i * NW
        nb0 = blk * NODE_BLK
        pltpu.sync_copy(ids_hbm.at[pl.ds(nb0 * nwords, NODE_BLK * nwords)],
                        ids_v)
        pltpu.sync_copy(w_hbm.at[pl.ds(nb0 * nwords, NODE_BLK * nwords)], w_v)

        issue_gather(0, 0)

        def batch_body(bt, par):
            pltpu.make_async_copy(table_hbm.at[ids_v.at[pl.ds(bt * bw, bw)]],
                                  rows[par].at[pl.ds(0, bw)],
                                  sems[par]).wait()

            @pl.when(bt + 1 < NBT)
            def _():
                issue_gather(bt + 1, 1 - par)

            def node_body(u, _):
                wbase = (bt * BN + u) * nwords
                rbase = u * nwords
                acc = [jnp.zeros((16,), jnp.float32) for _ in range(8)]
                for j in range(nwords):
                    wj = plsc.load_gather(w_v, [_full16(wbase + j)])
                    if packed:
                        for k in range(4):
                            vi = rows[par][rbase + j, pl.ds(k * 16, 16)]
                            v32 = plsc.bitcast(vi, jnp.bfloat16)
                            a, b = plsc.unpack(
                                v32, format=plsc.PackFormat.INTERLEAVED)
                            acc[2 * k] = acc[2 * k] + a * wj
                            acc[2 * k + 1] = acc[2 * k + 1] + b * wj
                    else:
                        for l in range(8):
                            acc[l] = (acc[l]
                                      + rows[par][rbase + j,
                                                  pl.ds(l * 16, 16)]
                                      * wj)
                for l in range(8):
                    obuf[u, pl.ds(l * 16, 16)] = acc[l]
                return _

            lax.fori_loop(0, BN, node_body, 0)
            pltpu.sync_copy(obuf, out_hbm.at[pl.ds(nb0 + bt * BN, BN)])

        def pair_body(p, _):
            batch_body(p * 2, 0)
            batch_body(p * 2 + 1, 1)
            return _

        lax.fori_loop(0, NBT // 2, pair_body, 0)
        if NBT % 2:
            batch_body(NBT - 1, 0)
        return _

    lax.fori_loop(0, nblk, block_body, 0)


def _emb_kernel_fn(text_ids, wt, label_ids, wl, word_tab, label_tab,
                   out_t, out_l,
                   ids_tv, wt_v, rt0, rt1, ids_lv, wl_v, obuf,
                   sem0, sem1):
    _emb_body(text_ids, wt, word_tab, out_t, ids_tv, wt_v, (rt0, rt1), obuf,
              (sem0, sem1), LT)
    _emb_body(label_ids, wl, label_tab, out_l, ids_lv, wl_v, (rt0, rt1), obuf,
              (sem0, sem1), LL)


@functools.lru_cache(maxsize=None)
def _emb_call():
    return functools.partial(
    pl.kernel,
    out_type=[jax.ShapeDtypeStruct((N, D), jnp.float32),
              jax.ShapeDtypeStruct((N, D), jnp.float32)],
    mesh=plsc.VectorSubcoreMesh(core_axis_name="c", subcore_axis_name="s"),
    compiler_params=pltpu.CompilerParams(needs_layout_passes=False),
    scratch_types=[
        pltpu.VMEM((NODE_BLK * LT,), jnp.int32),
        pltpu.VMEM((NODE_BLK * LT,), jnp.float32),
        pltpu.VMEM((BN * LT, D), jnp.float32),
        pltpu.VMEM((BN * LT, D), jnp.float32),
        pltpu.VMEM((NODE_BLK * LL,), jnp.int32),
        pltpu.VMEM((NODE_BLK * LL,), jnp.float32),
        pltpu.VMEM((BN, D), jnp.float32),
        pltpu.SemaphoreType.DMA,
        pltpu.SemaphoreType.DMA,
    ],
    )(_emb_kernel_fn)


NPH = 5
EPP = EPT // NPH
CPP = NCHUNK // NPH
NBUF = 4


def _msg_branch(src_hbm, dst4_hbm, ew_hbm, h_hbm, out_hbm, acc,
                src_flat, dst_loc, ew_flat, rows, gsem, ssem):
    s = lax.axis_index("s")

    base = s * ROWS_PT
    for ci in range(13):
        r0 = base + ci * 48
        pltpu.sync_copy(h_hbm.at[pl.ds(r0, 48)], rows[0].at[pl.ds(0, 48)])
        pltpu.sync_copy(rows[0].at[pl.ds(0, 48)], acc.at[pl.ds(r0, 48)])

    @pl.when(s == NS - 1)
    def _():
        r0 = NS * ROWS_PT
        pltpu.sync_copy(h_hbm.at[pl.ds(r0, N - NS * ROWS_PT)],
                        rows[1].at[pl.ds(0, N - NS * ROWS_PT)])
        pltpu.sync_copy(rows[1].at[pl.ds(0, N - NS * ROWS_PT)],
                        acc.at[pl.ds(r0, N - NS * ROWS_PT)])

    plsc.subcore_barrier()

    dst_s = dst4_hbm.at[s]

    def issue_gather(m, r):
        pltpu.async_copy(h_hbm.at[src_flat.at[pl.ds(m * EK, EK)]],
                         rows[r], gsem[r])

    def wait_gather(m, r):
        pltpu.make_async_copy(h_hbm.at[src_flat.at[pl.ds(m * EK, EK)]],
                              rows[r], gsem[r]).wait()

    def wait_scatter(m, r):
        pltpu.make_async_copy(rows[r], acc.at[dst_loc.at[m]], ssem[r]).wait()

    def chunk_body(m, r):
        @pl.when(m >= 2)
        def _():
            wait_scatter(m - 2, r ^ 2)

        @pl.when(m + 2 < CPP)
        def _():
            issue_gather(m + 2, r ^ 2)

        wait_gather(m, r)
        wb = m * EK

        def scale2(e2, _):
            e = e2 * 4
            for u in range(4):
                wv = plsc.load_gather(
                    ew_flat, [jnp.full((16,), wb + e + u, jnp.int32)])
                for l in range(8):
                    rows[r][e + u, pl.ds(l * 16, 16)] = (
                        rows[r][e + u, pl.ds(l * 16, 16)] * wv)
            return _

        lax.fori_loop(0, EK // 4, scale2, 0)

        pltpu.async_copy(rows[r], acc.at[dst_loc.at[m]], ssem[r], add=True)

    def phase_body(ph, _):
        e0 = s * EPT + ph * EPP
        pltpu.sync_copy(src_hbm.at[pl.ds(e0, EPP)], src_flat)
        pltpu.sync_copy(ew_hbm.at[pl.ds(e0, EPP)], ew_flat)
        pltpu.sync_copy(dst_s.at[ph], dst_loc)

        issue_gather(0, 0)
        issue_gather(1, 1)

        def iter_body(it, _):
            for u in range(NBUF):
                chunk_body(it * NBUF + u, u)
            return _

        lax.fori_loop(0, CPP // NBUF, iter_body, 0)
        wait_scatter(CPP - 2, (CPP - 2) % NBUF)
        wait_scatter(CPP - 1, (CPP - 1) % NBUF)
        return _

    lax.fori_loop(0, NPH, phase_body, 0)

    plsc.subcore_barrier()

    for ci in range(13):
        r0 = base + ci * 48
        pltpu.sync_copy(acc.at[pl.ds(r0, 48)], rows[0].at[pl.ds(0, 48)])
        pltpu.sync_copy(rows[0].at[pl.ds(0, 48)], out_hbm.at[pl.ds(r0, 48)])

    @pl.when(s == NS - 1)
    def _():
        r0 = NS * ROWS_PT
        pltpu.sync_copy(acc.at[pl.ds(r0, N - NS * ROWS_PT)],
                        rows[1].at[pl.ds(0, N - NS * ROWS_PT)])
        pltpu.sync_copy(rows[1].at[pl.ds(0, N - NS * ROWS_PT)],
                        out_hbm.at[pl.ds(r0, N - NS * ROWS_PT)])


def _msg_kernel_fn(src, dst4, ew, ht, hl, out_t, out_l,
                   acc, src_flat, dst_loc, ew_flat,
                   rg0, rg1, rg2, rg3,
                   gsem0, gsem1, gsem2, gsem3,
                   ssem0, ssem1, ssem2, ssem3):
    c = lax.axis_index("c")
    rows = (rg0, rg1, rg2, rg3)
    gsem = (gsem0, gsem1, gsem2, gsem3)
    ssem = (ssem0, ssem1, ssem2, ssem3)

    @pl.when(c == 0)
    def _():
        _msg_branch(src, dst4, ew, ht, out_t, acc,
                    src_flat, dst_loc, ew_flat, rows, gsem, ssem)

    @pl.when(c == 1)
    def _():
        _msg_branch(src, dst4, ew, hl, out_l, acc,
                    src_flat, dst_loc, ew_flat, rows, gsem, ssem)


@functools.lru_cache(maxsize=None)
def _msg_call():
    return functools.partial(
    pl.kernel,
    out_type=[jax.ShapeDtypeStruct((N, D), jnp.float32),
              jax.ShapeDtypeStruct((N, D), jnp.float32)],
    mesh=plsc.VectorSubcoreMesh(core_axis_name="c", subcore_axis_name="s"),
    compiler_params=pltpu.CompilerParams(needs_layout_passes=False),
    scratch_types=[
        pltpu.VMEM_SHARED((N, D), jnp.float32),
        pltpu.VMEM((EPP,), jnp.int32),
        pltpu.VMEM((CPP, EK), jnp.int32),
        pltpu.VMEM((EPP,), jnp.float32),
        pltpu.VMEM((EK, D), jnp.float32),
        pltpu.VMEM((EK, D), jnp.float32),
        pltpu.VMEM((EK, D), jnp.float32),
        pltpu.VMEM((EK, D), jnp.float32),
        pltpu.SemaphoreType.DMA,
        pltpu.SemaphoreType.DMA,
        pltpu.SemaphoreType.DMA,
        pltpu.SemaphoreType.DMA,
        pltpu.SemaphoreType.DMA,
        pltpu.SemaphoreType.DMA,
        pltpu.SemaphoreType.DMA,
        pltpu.SemaphoreType.DMA,
    ],
    )(_msg_kernel_fn)


C_BLK = 1000


def _combine_fn(at_ref, wt_ref, bt_ref, al_ref, wl_ref, bl_ref,
                ot_ref, ol_ref):
    xt = jnp.dot(at_ref[...], wt_ref[...],
                 preferred_element_type=jnp.float32) + bt_ref[...]
    ot_ref[...] = jnp.maximum(xt, 0.0)
    xl = jnp.dot(al_ref[...], wl_ref[...],
                 preferred_element_type=jnp.float32) + bl_ref[...]
    ol_ref[...] = jnp.maximum(xl, 0.0)


def _combine(acc_t, wtT, bt, acc_l, wlT, bl):
    blk = lambda i: (i, 0)
    fix = lambda i: (0, 0)
    return pl.pallas_call(
        _combine_fn,
        grid=(N // C_BLK,),
        in_specs=[
            pl.BlockSpec((C_BLK, D), blk),
            pl.BlockSpec((D, H), fix),
            pl.BlockSpec((1, H), fix),
            pl.BlockSpec((C_BLK, D), blk),
            pl.BlockSpec((D, H), fix),
            pl.BlockSpec((1, H), fix),
        ],
        out_specs=[
            pl.BlockSpec((C_BLK, H), blk),
            pl.BlockSpec((C_BLK, H), blk),
        ],
        out_shape=[jax.ShapeDtypeStruct((N, H), jnp.float32),
                   jax.ShapeDtypeStruct((N, H), jnp.float32)],
    )(acc_t, wtT, bt.reshape(1, H), acc_l, wlT, bl.reshape(1, H))


R_BLK = 1000
NR = N // R_BLK


def _readout_fn(at_ref, wt_ref, bt_ref, al_ref, wl_ref, bl_ref,
                g_ref, adap_ref, w1_ref, b1_ref,
                wo_ref, bo_ref, out_ref, acc_t, acc_l, cnt):
    i = pl.program_id(0)

    @pl.when(i == 0)
    def _():
        acc_t[...] = jnp.zeros_like(acc_t)
        acc_l[...] = jnp.zeros_like(acc_l)
        cnt[...] = jnp.zeros_like(cnt)

    ht = jnp.maximum(jnp.dot(at_ref[...], wt_ref[...],
                             preferred_element_type=jnp.float32)
                     + bt_ref[...], 0.0)
    hl = jnp.maximum(jnp.dot(al_ref[...], wl_ref[...],
                             preferred_element_type=jnp.float32)
                     + bl_ref[...], 0.0)

    g = g_ref[...]
    cols = lax.broadcasted_iota(jnp.int32, (R_BLK, 128), 1)
    onehot = (g == cols).astype(jnp.float32)

    contract = (((0,), (0,)), ((), ()))
    acc_t[...] += lax.dot_general(onehot, ht, contract,
                                  preferred_element_type=jnp.float32)
    acc_l[...] += lax.dot_general(onehot, hl, contract,
                                  preferred_element_type=jnp.float32)
    ones = jnp.ones((R_BLK, 1), jnp.float32)
    cnt[...] += lax.dot_general(onehot, ones, contract,
                                preferred_element_type=jnp.float32)

    @pl.when(i == NR - 1)
    def _():
        c = jnp.maximum(cnt[...], 1.0)
        r_t = acc_t[...] / c
        r_l = acc_l[...] / c
        a0 = adap_ref[0, 0]
        a1 = adap_ref[0, 1]
        adap_out = a0 * r_t + a1 * r_l
        fused = jnp.maximum(
            jnp.dot(adap_out, w1_ref[...],
                    preferred_element_type=jnp.float32) + b1_ref[...], 0.0)
        res = jnp.dot(fused, wo_ref[...],
                      preferred_element_type=jnp.float32) + bo_ref[...]
        out_ref[...] = res[:B, :]


def _readout(at2, wtT, bt, al2, wlT, bl, gids2, adap_W, w1T, b1, woT, bo):
    blk = lambda i: (i, 0)
    fix = lambda i: (0, 0)
    return pl.pallas_call(
        _readout_fn,
        grid=(NR,),
        in_specs=[
            pl.BlockSpec((R_BLK, H), blk),
            pl.BlockSpec((D, H), fix),
            pl.BlockSpec((1, H), fix),
            pl.BlockSpec((R_BLK, H), blk),
            pl.BlockSpec((D, H), fix),
            pl.BlockSpec((1, H), fix),
            pl.BlockSpec((R_BLK, 1), blk),
            pl.BlockSpec(memory_space=pltpu.SMEM),
            pl.BlockSpec((H, H), fix),
            pl.BlockSpec((1, H), fix),
            pl.BlockSpec((H, OUT), fix),
            pl.BlockSpec((1, OUT), fix),
        ],
        out_specs=pl.BlockSpec((B, OUT), fix),
        out_shape=jax.ShapeDtypeStruct((B, OUT), jnp.float32),
        scratch_shapes=[
            pltpu.VMEM((128, H), jnp.float32),
            pltpu.VMEM((128, H), jnp.float32),
            pltpu.VMEM((128, 1), jnp.float32),
        ],
    )(at2, wtT, bt.reshape(1, H), al2, wlT, bl.reshape(1, H), gids2,
      adap_W, w1T, b1.reshape(1, H), woT, bo.reshape(1, OUT))


def kernel(edge_index, edge_weight, text_ids, text_lens, label_ids,
           label_lens, graph_ids, embedding_Word, Emb_label,
           W_text_0, b_text_0, W_text_1, b_text_1,
           W_label_0, b_label_0, W_label_1, b_label_1,
           adap_W, fusion_W1, fusion_b1, fusion_Wo, fusion_bo):
    pad_idx = (jnp.arange(EPAD - E, dtype=jnp.int32) % N)
    src = jnp.concatenate([edge_index[0].astype(jnp.int32), pad_idx])
    dst = jnp.concatenate([edge_index[1].astype(jnp.int32), pad_idx])
    ew = jnp.concatenate([edge_weight.astype(jnp.float32),
                          jnp.zeros((EPAD - E,), jnp.float32)])

    tl = text_lens.astype(jnp.float32)
    wt = jnp.where(jnp.arange(LT)[None, :] < text_lens[:, None],
                   1.0 / tl[:, None], 0.0).astype(jnp.float32)
    wl = jnp.where(jnp.arange(LL)[None, :] < label_lens[:, None],
                   1.0, 0.0).astype(jnp.float32)

    text_feat, label_feat = _emb_call()(
        text_ids.astype(jnp.int32).reshape(-1), wt.reshape(-1),
        label_ids.astype(jnp.int32).reshape(-1), wl.reshape(-1),
        embedding_Word, Emb_label)

    dst4 = dst.reshape(NS, NPH, CPP, EK)
    acc_t, acc_l = _msg_call()(src, dst4, ew, text_feat, label_feat)
    h1t, h1l = _combine(acc_t, W_text_0.T, b_text_0,
                        acc_l, W_label_0.T, b_label_0)
    acc_t2, acc_l2 = _msg_call()(src, dst4, ew, h1t, h1l)

    res = _readout(acc_t2, W_text_1.T, b_text_1,
                   acc_l2, W_label_1.T, b_label_1,
                   graph_ids.astype(jnp.int32).reshape(N, 1),
                   adap_W, fusion_W1.T, fusion_b1, fusion_Wo.T, fusion_bo)
    return res
